# baseline (speedup 1.0000x reference)
import functools

import jax
import jax.numpy as jnp
from jax import lax
from jax.experimental import pallas as pl
from jax.experimental.pallas import tpu as pltpu
from jax.experimental.pallas import tpu_sc as plsc

_E = 320000
_T = 640000
_H = 128
_F = 64


def _silu(v):
    return v * jax.nn.sigmoid(v)


_BLK1 = 2560


def _s1_body(x_ref, rbf_ref, Wr1, Wr2, Wkj, bkj, Wji, bji, Wdn, xji_ref, xkj_ref):
    xb = x_ref[...]
    xji_ref[...] = _silu(xb @ Wji[...] + bji[...])
    rbf_e = (rbf_ref[...] @ Wr1[...]) @ Wr2[...]
    t = _silu(xb @ Wkj[...] + bkj[...]) * rbf_e
    v = _silu(t @ Wdn[...])
    xkj_ref[...] = jnp.concatenate([v, jnp.zeros_like(v)], axis=1)


def _stage1(x, rbf, Wr1, Wr2, Wkj, bkj, Wji, bji, Wdn):
    nb = _E // _BLK1
    full = lambda a: pl.BlockSpec(a.shape, lambda i: (0,) * a.ndim)
    return pl.pallas_call(
        _s1_body,
        grid=(nb,),
        in_specs=[
            pl.BlockSpec((_BLK1, _H), lambda i: (i, 0)),
            pl.BlockSpec((_BLK1, 6), lambda i: (i, 0)),
            full(Wr1), full(Wr2), full(Wkj), full(bkj), full(Wji), full(bji),
            full(Wdn),
        ],
        out_specs=[
            pl.BlockSpec((_BLK1, _H), lambda i: (i, 0)),
            pl.BlockSpec((_BLK1, 2 * _F), lambda i: (i, 0)),
        ],
        out_shape=[
            jax.ShapeDtypeStruct((_E, _H), jnp.float32),
            jax.ShapeDtypeStruct((_E, 2 * _F), jnp.float32),
        ],
    )(x, rbf, Wr1, Wr2, Wkj, bkj, Wji, bji, Wdn)


_BLK2 = 2560


def _s2_body(sbf_ref, Ws1, Ws2, out_ref):
    r = (sbf_ref[...] @ Ws1[...]) @ Ws2[...]
    out_ref[...] = jnp.concatenate([r, jnp.zeros_like(r)], axis=1)


def _stage2(sbf, Ws1, Ws2):
    nb = _T // _BLK2
    full = lambda a: pl.BlockSpec(a.shape, lambda i: (0,) * a.ndim)
    return pl.pallas_call(
        _s2_body,
        grid=(nb,),
        in_specs=[
            pl.BlockSpec((_BLK2, 42), lambda i: (i, 0)),
            full(Ws1), full(Ws2),
        ],
        out_specs=pl.BlockSpec((_BLK2, 2 * _F), lambda i: (i, 0)),
        out_shape=jax.ShapeDtypeStruct((_T, 2 * _F), jnp.float32),
    )(sbf, Ws1, Ws2)


_BLK4 = 2560


def _s4_body(seg_ref, xji_ref, x_ref, Wup, Wb1, bb1, Wb2, bb2, Wl, bl,
             Wa1, ba1, Wa2, ba2, Wa3, ba3, Wa4, ba4, out_ref):
    h = xji_ref[...] + _silu(seg_ref[...][:, :_F] @ Wup[...])
    h = h + _silu(_silu(h @ Wb1[...] + bb1[...]) @ Wb2[...] + bb2[...])
    h = _silu(h @ Wl[...] + bl[...]) + x_ref[...]
    h = h + _silu(_silu(h @ Wa1[...] + ba1[...]) @ Wa2[...] + ba2[...])
    h = h + _silu(_silu(h @ Wa3[...] + ba3[...]) @ Wa4[...] + ba4[...])
    out_ref[...] = h


def _stage4(seg, xji, x, Wup, Wb1, bb1, Wb2, bb2, Wl, bl,
            Wa1, ba1, Wa2, ba2, Wa3, ba3, Wa4, ba4):
    nb = _E // _BLK4
    full = lambda a: pl.BlockSpec(a.shape, lambda i: (0,) * a.ndim)
    ws = [Wup, Wb1, bb1, Wb2, bb2, Wl, bl, Wa1, ba1, Wa2, ba2, Wa3, ba3, Wa4, ba4]
    return pl.pallas_call(
        _s4_body,
        grid=(nb,),
        in_specs=[
            pl.BlockSpec((_BLK4, 2 * _F), lambda i: (i, 0)),
            pl.BlockSpec((_BLK4, _H), lambda i: (i, 0)),
            pl.BlockSpec((_BLK4, _H), lambda i: (i, 0)),
        ] + [full(w) for w in ws],
        out_specs=pl.BlockSpec((_BLK4, _H), lambda i: (i, 0)),
        out_shape=jax.ShapeDtypeStruct((_E, _H), jnp.float32),
    )(seg, xji, x, *ws)


def _make_sc_scatter(E, T, F, NB, CH, G, interpret=False, debug_level=9):
    info_nc, info_ns = 2, 16
    RPB = E // NB
    assert RPB * NB == E and RPB % 16 == 0
    PASSES = NB // info_nc
    ACC = RPB + 16
    while max(d for d in range(1, 97) if (ACC // info_ns) % d == 0) < 32:
        ACC += 16
    TSPAN = T // info_ns
    assert TSPAN % CH == 0
    NCH = TSPAN // CH
    assert CH % 16 == 0 and G % 16 == 0
    GB = G + 16
    NV = CH // 16
    ZPT = ACC // info_ns
    zr = 1
    for d in range(2, 97):
        if ZPT % d == 0:
            zr = d
    ZR = zr
    NZ = ZPT // ZR
    CPT = (RPB // info_ns) & ~7
    CPL = RPB - CPT * (info_ns - 1)
    assert CPL % 8 == 0 and CPL >= 0

    mesh = plsc.VectorSubcoreMesh(core_axis_name="c", subcore_axis_name="s",
                                  num_cores=info_nc, num_subcores=info_ns)

    @functools.partial(
        pl.kernel,
        out_type=jax.ShapeDtypeStruct((E, 2 * F), jnp.float32),
        mesh=mesh,
        interpret=interpret,
        compiler_params=pltpu.CompilerParams(needs_layout_passes=False),
        scratch_types=[
            pltpu.VMEM((CH,), jnp.int32),
            pltpu.VMEM((CH,), jnp.int32),
            pltpu.VMEM((GB,), jnp.int32),
            pltpu.VMEM((GB,), jnp.int32),
            pltpu.VMEM((GB,), jnp.int32),
            pltpu.VMEM((GB, 2 * F), jnp.float32),
            pltpu.VMEM((GB, 2 * F), jnp.float32),
            pltpu.VMEM((ZR, 2 * F), jnp.float32),
            pltpu.VMEM_SHARED((ACC, 2 * F), jnp.float32),
            pltpu.SemaphoreType.DMA,
            pltpu.SemaphoreType.DMA,
        ],
    )
    def sc_fn(xkj_hbm, sbf_hbm, idxkj_hbm, idxji_hbm, out_hbm,
              jib, kjb, ckj, ct, cloc,
              xrows, srows, zbuf, acc, sem1, sem2):
        c = lax.axis_index("c")
        s = lax.axis_index("s")
        zero16 = jnp.zeros((16,), jnp.float32)
        iota16 = lax.iota(jnp.int32, 16)
        sent_row = jnp.zeros((16,), jnp.int32) + (RPB + s)
        sent_idx = iota16 + s * 16

        if debug_level >= -1:
            def zb_fill(i, _):
                for q in range(2 * F // 16):
                    zbuf[i, pl.ds(q * 16, 16)] = zero16
                return 0
            lax.fori_loop(0, ZR, zb_fill, 0)

        def flush(cnt):
            def padf(j, _):
                o = jnp.minimum(cnt + j * 16, G)
                ckj[pl.ds(o, 16)] = sent_idx
                ct[pl.ds(o, 16)] = sent_idx
                cloc[pl.ds(o, 16)] = sent_row
                return 0
            lax.fori_loop(0, G // 16 + 1, padf, 0)

            d1 = pltpu.async_copy(xkj_hbm.at[ckj], xrows, sem1)
            d2 = pltpu.async_copy(sbf_hbm.at[ct], srows, sem2)
            d1.wait()
            d2.wait()

            def mult(r, _):
                for q in range(2 * F // 16):
                    xrows[r, pl.ds(q * 16, 16)] = (
                        xrows[r, pl.ds(q * 16, 16)]
                        * srows[r, pl.ds(q * 16, 16)])
                return 0
            lax.fori_loop(0, GB, mult, 0)
            pltpu.sync_copy(xrows, acc.at[cloc], add=True)

        def one_pass(p, _):
            b = c * PASSES + p
            lo = b * RPB
            if debug_level >= 0:
                for z in range(NZ):
                    pltpu.sync_copy(zbuf, acc.at[pl.ds(s * ZPT + z * ZR, ZR)])
            plsc.subcore_barrier()

            def one_chunk(ch, cnt):
                off = s * TSPAN + ch * CH
                pltpu.sync_copy(idxji_hbm.at[pl.ds(off, CH)], jib)
                pltpu.sync_copy(idxkj_hbm.at[pl.ds(off, CH)], kjb)
                if debug_level < 3:
                    return cnt

                def compact(v, cnt):
                    vji = jib[pl.ds(v * 16, 16)]
                    vkj = kjb[pl.ds(v * 16, 16)]
                    msk = (vji >= lo) & (vji < lo + RPB)
                    loc = vji - lo
                    tv = off + v * 16 + iota16
                    plsc.store_compressed(ckj.at[pl.ds(cnt, 16)], vkj, mask=msk)
                    plsc.store_compressed(ct.at[pl.ds(cnt, 16)], tv, mask=msk)
                    plsc.store_compressed(cloc.at[pl.ds(cnt, 16)], loc, mask=msk)
                    cnt = cnt + jnp.sum(msk.astype(jnp.int32))

                    def do_flush(cc):
                        if debug_level >= 4:
                            flush(cc)
                        return jnp.int32(0)
                    return lax.cond(cnt >= G - 15, do_flush,
                                    lambda cc: cc, cnt)
                return lax.fori_loop(0, NV, compact, cnt)
            if debug_level >= 2:
                cnt = lax.fori_loop(0, NCH, one_chunk, jnp.int32(0))
                if debug_level >= 4:
                    flush(cnt)

            plsc.subcore_barrier()
            if debug_level >= 1:
                @pl.when(s < info_ns - 1)
                def _():
                    pltpu.sync_copy(acc.at[pl.ds(s * CPT, CPT)],
                                    out_hbm.at[pl.ds(b * RPB + s * CPT, CPT)])

                @pl.when(s == info_ns - 1)
                def _():
                    pltpu.sync_copy(
                        acc.at[pl.ds((info_ns - 1) * CPT, CPL)],
                        out_hbm.at[pl.ds(b * RPB + (info_ns - 1) * CPT, CPL)])
            plsc.subcore_barrier()
            return 0
        lax.fori_loop(0, PASSES, one_pass, 0)

    return sc_fn


_sc_scatter = None
_DBG = 9


def _get_sc_scatter():
    global _sc_scatter
    if _sc_scatter is None:
        _sc_scatter = _make_sc_scatter(_E, _T, _F, NB=32, CH=2000, G=96,
                                       debug_level=_DBG)
    return _sc_scatter


def kernel(x, rbf, sbf, idx_kj, idx_ji, W_rbf1, W_rbf2, W_sbf1, W_sbf2,
           W_kj, b_kj, W_ji, b_ji, W_down, W_up,
           W_bs1, b_bs1, W_bs2, b_bs2, W_lin, b_lin,
           W_as1, b_as1, W_as2, b_as2, W_as3, b_as3, W_as4, b_as4):
    r2 = lambda v: v.reshape(1, -1)
    xji, xkj = _stage1(x, rbf, W_rbf1, W_rbf2, W_kj, r2(b_kj), W_ji,
                       r2(b_ji), W_down)
    sbf_e = _stage2(sbf, W_sbf1, W_sbf2)
    seg = _get_sc_scatter()(xkj, sbf_e, idx_kj.astype(jnp.int32),
                            idx_ji.astype(jnp.int32))
    if _DBG < 9:
        fix = jax.ops.segment_sum(
            xkj[idx_kj, :_F] * sbf_e[:, :_F], idx_ji, num_segments=_E)
        seg = seg.at[:, :_F].add(fix)
    return _stage4(seg, xji, x, W_up, W_bs1, r2(b_bs1), W_bs2, r2(b_bs2),
                   W_lin, r2(b_lin), W_as1, r2(b_as1), W_as2, r2(b_as2),
                   W_as3, r2(b_as3), W_as4, r2(b_as4))


# speedup vs baseline: 1.0879x; 1.0879x over previous
import functools

import jax
import jax.numpy as jnp
from jax import lax
from jax.experimental import pallas as pl
from jax.experimental.pallas import tpu as pltpu
from jax.experimental.pallas import tpu_sc as plsc

_E = 320000
_T = 640000
_H = 128
_F = 64


def _silu(v):
    return v * jax.nn.sigmoid(v)


_BLK1 = 2560


def _s1_body(x_ref, rbf_ref, Wr1, Wr2, Wkj, bkj, Wji, bji, Wdn, xji_ref, xkj_ref):
    xb = x_ref[...]
    xji_ref[...] = _silu(xb @ Wji[...] + bji[...])
    rbf_e = (rbf_ref[...] @ Wr1[...]) @ Wr2[...]
    t = _silu(xb @ Wkj[...] + bkj[...]) * rbf_e
    v = _silu(t @ Wdn[...])
    xkj_ref[...] = jnp.concatenate([v, jnp.zeros_like(v)], axis=1)


def _stage1(x, rbf, Wr1, Wr2, Wkj, bkj, Wji, bji, Wdn):
    nb = _E // _BLK1
    full = lambda a: pl.BlockSpec(a.shape, lambda i: (0,) * a.ndim)
    return pl.pallas_call(
        _s1_body,
        grid=(nb,),
        in_specs=[
            pl.BlockSpec((_BLK1, _H), lambda i: (i, 0)),
            pl.BlockSpec((_BLK1, 6), lambda i: (i, 0)),
            full(Wr1), full(Wr2), full(Wkj), full(bkj), full(Wji), full(bji),
            full(Wdn),
        ],
        out_specs=[
            pl.BlockSpec((_BLK1, _H), lambda i: (i, 0)),
            pl.BlockSpec((_BLK1, 2 * _F), lambda i: (i, 0)),
        ],
        out_shape=[
            jax.ShapeDtypeStruct((_E, _H), jnp.float32),
            jax.ShapeDtypeStruct((_E, 2 * _F), jnp.float32),
        ],
    )(x, rbf, Wr1, Wr2, Wkj, bkj, Wji, bji, Wdn)


_BLK2 = 2560


def _s2_body(sbf_ref, Ws1, Ws2, out_ref):
    r = (sbf_ref[...] @ Ws1[...]) @ Ws2[...]
    out_ref[...] = jnp.concatenate([r, jnp.zeros_like(r)], axis=1)


def _stage2(sbf, Ws1, Ws2):
    nb = _T // _BLK2
    full = lambda a: pl.BlockSpec(a.shape, lambda i: (0,) * a.ndim)
    return pl.pallas_call(
        _s2_body,
        grid=(nb,),
        in_specs=[
            pl.BlockSpec((_BLK2, 42), lambda i: (i, 0)),
            full(Ws1), full(Ws2),
        ],
        out_specs=pl.BlockSpec((_BLK2, 2 * _F), lambda i: (i, 0)),
        out_shape=jax.ShapeDtypeStruct((_T, 2 * _F), jnp.float32),
    )(sbf, Ws1, Ws2)


_BLK4 = 2560


def _s4_body(seg_ref, xji_ref, x_ref, Wup, Wb1, bb1, Wb2, bb2, Wl, bl,
             Wa1, ba1, Wa2, ba2, Wa3, ba3, Wa4, ba4, out_ref):
    h = xji_ref[...] + _silu(seg_ref[...][:, :_F] @ Wup[...])
    h = h + _silu(_silu(h @ Wb1[...] + bb1[...]) @ Wb2[...] + bb2[...])
    h = _silu(h @ Wl[...] + bl[...]) + x_ref[...]
    h = h + _silu(_silu(h @ Wa1[...] + ba1[...]) @ Wa2[...] + ba2[...])
    h = h + _silu(_silu(h @ Wa3[...] + ba3[...]) @ Wa4[...] + ba4[...])
    out_ref[...] = h


def _stage4(seg, xji, x, Wup, Wb1, bb1, Wb2, bb2, Wl, bl,
            Wa1, ba1, Wa2, ba2, Wa3, ba3, Wa4, ba4):
    nb = _E // _BLK4
    full = lambda a: pl.BlockSpec(a.shape, lambda i: (0,) * a.ndim)
    ws = [Wup, Wb1, bb1, Wb2, bb2, Wl, bl, Wa1, ba1, Wa2, ba2, Wa3, ba3, Wa4, ba4]
    return pl.pallas_call(
        _s4_body,
        grid=(nb,),
        in_specs=[
            pl.BlockSpec((_BLK4, 2 * _F), lambda i: (i, 0)),
            pl.BlockSpec((_BLK4, _H), lambda i: (i, 0)),
            pl.BlockSpec((_BLK4, _H), lambda i: (i, 0)),
        ] + [full(w) for w in ws],
        out_specs=pl.BlockSpec((_BLK4, _H), lambda i: (i, 0)),
        out_shape=jax.ShapeDtypeStruct((_E, _H), jnp.float32),
    )(seg, xji, x, *ws)


def _make_sc_scatter(E, T, F, NB, CH, G, interpret=False, debug_level=9):
    info_nc, info_ns = 2, 16
    RPB = E // NB
    assert RPB * NB == E and RPB % 16 == 0
    PASSES = NB // info_nc
    ACC = RPB + 16
    while max(d for d in range(1, 97) if (ACC // info_ns) % d == 0) < 32:
        ACC += 16
    TSPAN = T // info_ns
    assert TSPAN % CH == 0
    NCH = TSPAN // CH
    assert CH % 16 == 0 and G % 16 == 0
    GB = G + 16
    NV = CH // 16
    ZPT = ACC // info_ns
    zr = 1
    for d in range(2, 97):
        if ZPT % d == 0:
            zr = d
    ZR = zr
    NZ = ZPT // ZR
    CPT = (RPB // info_ns) & ~7
    CPL = RPB - CPT * (info_ns - 1)
    assert CPL % 8 == 0 and CPL >= 0

    mesh = plsc.VectorSubcoreMesh(core_axis_name="c", subcore_axis_name="s",
                                  num_cores=info_nc, num_subcores=info_ns)

    @functools.partial(
        pl.kernel,
        out_type=jax.ShapeDtypeStruct((E, 2 * F), jnp.float32),
        mesh=mesh,
        interpret=interpret,
        compiler_params=pltpu.CompilerParams(needs_layout_passes=False),
        scratch_types=[
            pltpu.VMEM((CH,), jnp.int32),
            pltpu.VMEM((CH,), jnp.int32),
            pltpu.VMEM((GB,), jnp.int32),
            pltpu.VMEM((GB,), jnp.int32),
            pltpu.VMEM((GB,), jnp.int32),
            pltpu.VMEM((GB, 2 * F), jnp.float32),
            pltpu.VMEM((GB, 2 * F), jnp.float32),
            pltpu.VMEM((ZR, 2 * F), jnp.float32),
            pltpu.VMEM_SHARED((ACC, 2 * F), jnp.float32),
            pltpu.SemaphoreType.DMA,
            pltpu.SemaphoreType.DMA,
        ],
    )
    def sc_fn(xkj_hbm, sbf_hbm, idxkj_hbm, idxji_hbm, out_hbm,
              jib, kjb, ckj, ct, cloc,
              xrows, srows, zbuf, acc, sem1, sem2):
        c = lax.axis_index("c")
        s = lax.axis_index("s")
        zero16 = jnp.zeros((16,), jnp.float32)
        iota16 = lax.iota(jnp.int32, 16)
        sent_row = jnp.zeros((16,), jnp.int32) + (RPB + s)
        sent_idx = iota16 + s * 16

        if debug_level >= -1:
            def zb_fill(i, _):
                for q in range(2 * F // 16):
                    zbuf[i, pl.ds(q * 16, 16)] = zero16
                return 0
            lax.fori_loop(0, ZR, zb_fill, 0)

        def flush(cnt):
            def padf(j, _):
                o = jnp.minimum(cnt + j * 16, G)
                ckj[pl.ds(o, 16)] = sent_idx
                ct[pl.ds(o, 16)] = sent_idx
                cloc[pl.ds(o, 16)] = sent_row
                return 0
            lax.fori_loop(0, G // 16 + 1, padf, 0)

            d1 = pltpu.async_copy(xkj_hbm.at[ckj], xrows, sem1)
            d2 = pltpu.async_copy(sbf_hbm.at[ct], srows, sem2)
            d1.wait()
            d2.wait()

            def mult(r, _):
                for q in range(F // 16):
                    xrows[r, pl.ds(q * 16, 16)] = (
                        xrows[r, pl.ds(q * 16, 16)]
                        * srows[r, pl.ds(q * 16, 16)])
                return 0
            lax.fori_loop(0, GB, mult, 0)
            pltpu.sync_copy(xrows, acc.at[cloc], add=True)

        def one_pass(p, _):
            b = c * PASSES + p
            lo = b * RPB
            if debug_level >= 0:
                for z in range(NZ):
                    pltpu.sync_copy(zbuf, acc.at[pl.ds(s * ZPT + z * ZR, ZR)])
            plsc.subcore_barrier()

            def one_chunk(ch, cnt):
                off = s * TSPAN + ch * CH
                pltpu.sync_copy(idxji_hbm.at[pl.ds(off, CH)], jib)
                pltpu.sync_copy(idxkj_hbm.at[pl.ds(off, CH)], kjb)
                if debug_level < 3:
                    return cnt

                def compact(v, cnt):
                    vji = jib[pl.ds(v * 16, 16)]
                    vkj = kjb[pl.ds(v * 16, 16)]
                    msk = (vji >= lo) & (vji < lo + RPB)
                    loc = vji - lo
                    tv = off + v * 16 + iota16
                    plsc.store_compressed(ckj.at[pl.ds(cnt, 16)], vkj, mask=msk)
                    plsc.store_compressed(ct.at[pl.ds(cnt, 16)], tv, mask=msk)
                    plsc.store_compressed(cloc.at[pl.ds(cnt, 16)], loc, mask=msk)
                    cnt = cnt + jnp.sum(msk.astype(jnp.int32))

                    def do_flush(cc):
                        if debug_level >= 4:
                            flush(cc)
                        return jnp.int32(0)
                    return lax.cond(cnt >= G - 15, do_flush,
                                    lambda cc: cc, cnt)
                return lax.fori_loop(0, NV, compact, cnt)
            if debug_level >= 2:
                cnt = lax.fori_loop(0, NCH, one_chunk, jnp.int32(0))
                if debug_level >= 4:
                    flush(cnt)

            plsc.subcore_barrier()
            if debug_level >= 1:
                @pl.when(s < info_ns - 1)
                def _():
                    pltpu.sync_copy(acc.at[pl.ds(s * CPT, CPT)],
                                    out_hbm.at[pl.ds(b * RPB + s * CPT, CPT)])

                @pl.when(s == info_ns - 1)
                def _():
                    pltpu.sync_copy(
                        acc.at[pl.ds((info_ns - 1) * CPT, CPL)],
                        out_hbm.at[pl.ds(b * RPB + (info_ns - 1) * CPT, CPL)])
            plsc.subcore_barrier()
            return 0
        lax.fori_loop(0, PASSES, one_pass, 0)

    return sc_fn


_sc_scatter = None
_DBG = 9


def _get_sc_scatter():
    global _sc_scatter
    if _sc_scatter is None:
        _sc_scatter = _make_sc_scatter(_E, _T, _F, NB=32, CH=4000, G=96,
                                       debug_level=_DBG)
    return _sc_scatter


def kernel(x, rbf, sbf, idx_kj, idx_ji, W_rbf1, W_rbf2, W_sbf1, W_sbf2,
           W_kj, b_kj, W_ji, b_ji, W_down, W_up,
           W_bs1, b_bs1, W_bs2, b_bs2, W_lin, b_lin,
           W_as1, b_as1, W_as2, b_as2, W_as3, b_as3, W_as4, b_as4):
    r2 = lambda v: v.reshape(1, -1)
    xji, xkj = _stage1(x, rbf, W_rbf1, W_rbf2, W_kj, r2(b_kj), W_ji,
                       r2(b_ji), W_down)
    sbf_e = _stage2(sbf, W_sbf1, W_sbf2)
    seg = _get_sc_scatter()(xkj, sbf_e, idx_kj.astype(jnp.int32),
                            idx_ji.astype(jnp.int32))
    if _DBG < 9:
        fix = jax.ops.segment_sum(
            xkj[idx_kj, :_F] * sbf_e[:, :_F], idx_ji, num_segments=_E)
        seg = seg.at[:, :_F].add(fix)
    return _stage4(seg, xji, x, W_up, W_bs1, r2(b_bs1), W_bs2, r2(b_bs2),
                   W_lin, r2(b_lin), W_as1, r2(b_as1), W_as2, r2(b_as2),
                   W_as3, r2(b_as3), W_as4, r2(b_as4))


# speedup vs baseline: 1.1319x; 1.0404x over previous
import functools

import jax
import jax.numpy as jnp
from jax import lax
from jax.experimental import pallas as pl
from jax.experimental.pallas import tpu as pltpu
from jax.experimental.pallas import tpu_sc as plsc

_E = 320000
_T = 640000
_H = 128
_F = 64


def _silu(v):
    return v * jax.nn.sigmoid(v)


_BLK1 = 2560


def _s1_body(x_ref, rbf_ref, Wr1, Wr2, Wkj, bkj, Wji, bji, Wdn, xji_ref, xkj_ref):
    xb = x_ref[...]
    xji_ref[...] = _silu(xb @ Wji[...] + bji[...])
    rbf_e = (rbf_ref[...] @ Wr1[...]) @ Wr2[...]
    t = _silu(xb @ Wkj[...] + bkj[...]) * rbf_e
    v = _silu(t @ Wdn[...])
    xkj_ref[...] = jnp.concatenate([v, jnp.zeros_like(v)], axis=1)


def _stage1(x, rbf, Wr1, Wr2, Wkj, bkj, Wji, bji, Wdn):
    nb = _E // _BLK1
    full = lambda a: pl.BlockSpec(a.shape, lambda i: (0,) * a.ndim)
    return pl.pallas_call(
        _s1_body,
        grid=(nb,),
        in_specs=[
            pl.BlockSpec((_BLK1, _H), lambda i: (i, 0)),
            pl.BlockSpec((_BLK1, 6), lambda i: (i, 0)),
            full(Wr1), full(Wr2), full(Wkj), full(bkj), full(Wji), full(bji),
            full(Wdn),
        ],
        out_specs=[
            pl.BlockSpec((_BLK1, _H), lambda i: (i, 0)),
            pl.BlockSpec((_BLK1, 2 * _F), lambda i: (i, 0)),
        ],
        out_shape=[
            jax.ShapeDtypeStruct((_E, _H), jnp.float32),
            jax.ShapeDtypeStruct((_E, 2 * _F), jnp.float32),
        ],
    )(x, rbf, Wr1, Wr2, Wkj, bkj, Wji, bji, Wdn)


_BLK2 = 2560


def _s2_body(sbf_ref, Ws1, Ws2, out_ref):
    r = (sbf_ref[...] @ Ws1[...]) @ Ws2[...]
    out_ref[...] = jnp.concatenate([r, jnp.zeros_like(r)], axis=1)


def _stage2(sbf, Ws1, Ws2):
    nb = _T // _BLK2
    full = lambda a: pl.BlockSpec(a.shape, lambda i: (0,) * a.ndim)
    return pl.pallas_call(
        _s2_body,
        grid=(nb,),
        in_specs=[
            pl.BlockSpec((_BLK2, 42), lambda i: (i, 0)),
            full(Ws1), full(Ws2),
        ],
        out_specs=pl.BlockSpec((_BLK2, 2 * _F), lambda i: (i, 0)),
        out_shape=jax.ShapeDtypeStruct((_T, 2 * _F), jnp.float32),
    )(sbf, Ws1, Ws2)


_BLK4 = 2560


def _bmm(a, w):
    return jax.lax.dot(a.astype(jnp.bfloat16), w.astype(jnp.bfloat16),
                       preferred_element_type=jnp.float32)


def _s4_body(seg_ref, xji_ref, x_ref, Wup, Wb1, bb1, Wb2, bb2, Wl, bl,
             Wa1, ba1, Wa2, ba2, Wa3, ba3, Wa4, ba4, out_ref):
    h = xji_ref[...] + _silu(_bmm(seg_ref[...][:, :_F], Wup[...]))
    h = h + _silu(_bmm(_silu(_bmm(h, Wb1[...]) + bb1[...]), Wb2[...]) + bb2[...])
    h = _silu(_bmm(h, Wl[...]) + bl[...]) + x_ref[...]
    h = h + _silu(_bmm(_silu(_bmm(h, Wa1[...]) + ba1[...]), Wa2[...]) + ba2[...])
    h = h + _silu(_bmm(_silu(_bmm(h, Wa3[...]) + ba3[...]), Wa4[...]) + ba4[...])
    out_ref[...] = h


def _stage4(seg, xji, x, Wup, Wb1, bb1, Wb2, bb2, Wl, bl,
            Wa1, ba1, Wa2, ba2, Wa3, ba3, Wa4, ba4):
    nb = _E // _BLK4
    full = lambda a: pl.BlockSpec(a.shape, lambda i: (0,) * a.ndim)
    ws = [Wup, Wb1, bb1, Wb2, bb2, Wl, bl, Wa1, ba1, Wa2, ba2, Wa3, ba3, Wa4, ba4]
    return pl.pallas_call(
        _s4_body,
        grid=(nb,),
        in_specs=[
            pl.BlockSpec((_BLK4, 2 * _F), lambda i: (i, 0)),
            pl.BlockSpec((_BLK4, _H), lambda i: (i, 0)),
            pl.BlockSpec((_BLK4, _H), lambda i: (i, 0)),
        ] + [full(w) for w in ws],
        out_specs=pl.BlockSpec((_BLK4, _H), lambda i: (i, 0)),
        out_shape=jax.ShapeDtypeStruct((_E, _H), jnp.float32),
    )(seg, xji, x, *ws)


def _make_sc_scatter(E, T, F, NB, CH, G, interpret=False, debug_level=9):
    info_nc, info_ns = 2, 16
    RPB = E // NB
    assert RPB * NB == E and RPB % 16 == 0
    PASSES = NB // info_nc
    ACC = RPB + 16
    while max(d for d in range(1, 97) if (ACC // info_ns) % d == 0) < 32:
        ACC += 16
    TSPAN = T // info_ns
    assert TSPAN % CH == 0
    NCH = TSPAN // CH
    assert CH % 16 == 0 and G % 16 == 0
    GB = G + 16
    NV = CH // 16
    ZPT = ACC // info_ns
    zr = 1
    for d in range(2, 97):
        if ZPT % d == 0:
            zr = d
    ZR = zr
    NZ = ZPT // ZR
    CPT = (RPB // info_ns) & ~7
    CPL = RPB - CPT * (info_ns - 1)
    assert CPL % 8 == 0 and CPL >= 0

    mesh = plsc.VectorSubcoreMesh(core_axis_name="c", subcore_axis_name="s",
                                  num_cores=info_nc, num_subcores=info_ns)

    @functools.partial(
        pl.kernel,
        out_type=jax.ShapeDtypeStruct((E, 2 * F), jnp.float32),
        mesh=mesh,
        interpret=interpret,
        compiler_params=pltpu.CompilerParams(needs_layout_passes=False),
        scratch_types=[
            pltpu.VMEM((CH,), jnp.int32),
            pltpu.VMEM((CH,), jnp.int32),
            pltpu.VMEM((GB,), jnp.int32),
            pltpu.VMEM((GB,), jnp.int32),
            pltpu.VMEM((GB,), jnp.int32),
            pltpu.VMEM((GB, 2 * F), jnp.float32),
            pltpu.VMEM((GB, 2 * F), jnp.float32),
            pltpu.VMEM((ZR, 2 * F), jnp.float32),
            pltpu.VMEM_SHARED((ACC, 2 * F), jnp.float32),
            pltpu.SemaphoreType.DMA,
            pltpu.SemaphoreType.DMA,
        ],
    )
    def sc_fn(xkj_hbm, sbf_hbm, idxkj_hbm, idxji_hbm, out_hbm,
              jib, kjb, ckj, ct, cloc,
              xrows, srows, zbuf, acc, sem1, sem2):
        c = lax.axis_index("c")
        s = lax.axis_index("s")
        zero16 = jnp.zeros((16,), jnp.float32)
        iota16 = lax.iota(jnp.int32, 16)
        sent_row = jnp.zeros((16,), jnp.int32) + (RPB + s)
        sent_idx = iota16 + s * 16

        if debug_level >= -1:
            def zb_fill(i, _):
                for q in range(2 * F // 16):
                    zbuf[i, pl.ds(q * 16, 16)] = zero16
                return 0
            lax.fori_loop(0, ZR, zb_fill, 0)

        def flush(cnt):
            def padf(j, _):
                o = jnp.minimum(cnt + j * 16, G)
                ckj[pl.ds(o, 16)] = sent_idx
                ct[pl.ds(o, 16)] = sent_idx
                cloc[pl.ds(o, 16)] = sent_row
                return 0
            lax.fori_loop(0, G // 16 + 1, padf, 0)

            d1 = pltpu.async_copy(xkj_hbm.at[ckj], xrows, sem1)
            d2 = pltpu.async_copy(sbf_hbm.at[ct], srows, sem2)
            d1.wait()
            d2.wait()

            def mult(r, _):
                for q in range(F // 16):
                    xrows[r, pl.ds(q * 16, 16)] = (
                        xrows[r, pl.ds(q * 16, 16)]
                        * srows[r, pl.ds(q * 16, 16)])
                return 0
            lax.fori_loop(0, GB, mult, 0)
            pltpu.sync_copy(xrows, acc.at[cloc], add=True)

        def one_pass(p, _):
            b = c * PASSES + p
            lo = b * RPB
            if debug_level >= 0:
                for z in range(NZ):
                    pltpu.sync_copy(zbuf, acc.at[pl.ds(s * ZPT + z * ZR, ZR)])
            plsc.subcore_barrier()

            def one_chunk(ch, cnt):
                off = s * TSPAN + ch * CH
                pltpu.sync_copy(idxji_hbm.at[pl.ds(off, CH)], jib)
                pltpu.sync_copy(idxkj_hbm.at[pl.ds(off, CH)], kjb)
                if debug_level < 3:
                    return cnt

                def compact(v, cnt):
                    vji = jib[pl.ds(v * 16, 16)]
                    vkj = kjb[pl.ds(v * 16, 16)]
                    msk = (vji >= lo) & (vji < lo + RPB)
                    loc = vji - lo
                    tv = off + v * 16 + iota16
                    plsc.store_compressed(ckj.at[pl.ds(cnt, 16)], vkj, mask=msk)
                    plsc.store_compressed(ct.at[pl.ds(cnt, 16)], tv, mask=msk)
                    plsc.store_compressed(cloc.at[pl.ds(cnt, 16)], loc, mask=msk)
                    cnt = cnt + plsc.all_reduce_population_count(msk)[0]

                    def do_flush(cc):
                        if debug_level >= 4:
                            flush(cc)
                        return jnp.int32(0)
                    return lax.cond(cnt >= G - 15, do_flush,
                                    lambda cc: cc, cnt)
                return lax.fori_loop(0, NV, compact, cnt)
            if debug_level >= 2:
                cnt = lax.fori_loop(0, NCH, one_chunk, jnp.int32(0))
                if debug_level >= 4:
                    flush(cnt)

            plsc.subcore_barrier()
            if debug_level >= 1:
                @pl.when(s < info_ns - 1)
                def _():
                    pltpu.sync_copy(acc.at[pl.ds(s * CPT, CPT)],
                                    out_hbm.at[pl.ds(b * RPB + s * CPT, CPT)])

                @pl.when(s == info_ns - 1)
                def _():
                    pltpu.sync_copy(
                        acc.at[pl.ds((info_ns - 1) * CPT, CPL)],
                        out_hbm.at[pl.ds(b * RPB + (info_ns - 1) * CPT, CPL)])
            plsc.subcore_barrier()
            return 0
        lax.fori_loop(0, PASSES, one_pass, 0)

    return sc_fn


_sc_scatter = None
_DBG = 9


def _get_sc_scatter():
    global _sc_scatter
    if _sc_scatter is None:
        _sc_scatter = _make_sc_scatter(_E, _T, _F, NB=32, CH=4000, G=96,
                                       debug_level=_DBG)
    return _sc_scatter


def kernel(x, rbf, sbf, idx_kj, idx_ji, W_rbf1, W_rbf2, W_sbf1, W_sbf2,
           W_kj, b_kj, W_ji, b_ji, W_down, W_up,
           W_bs1, b_bs1, W_bs2, b_bs2, W_lin, b_lin,
           W_as1, b_as1, W_as2, b_as2, W_as3, b_as3, W_as4, b_as4):
    r2 = lambda v: v.reshape(1, -1)
    xji, xkj = _stage1(x, rbf, W_rbf1, W_rbf2, W_kj, r2(b_kj), W_ji,
                       r2(b_ji), W_down)
    sbf_e = _stage2(sbf, W_sbf1, W_sbf2)
    seg = _get_sc_scatter()(xkj, sbf_e, idx_kj.astype(jnp.int32),
                            idx_ji.astype(jnp.int32))
    if _DBG < 9:
        fix = jax.ops.segment_sum(
            xkj[idx_kj, :_F] * sbf_e[:, :_F], idx_ji, num_segments=_E)
        seg = seg.at[:, :_F].add(fix)
    return _stage4(seg, xji, x, W_up, W_bs1, r2(b_bs1), W_bs2, r2(b_bs2),
                   W_lin, r2(b_lin), W_as1, r2(b_as1), W_as2, r2(b_as2),
                   W_as3, r2(b_as3), W_as4, r2(b_as4))


# speedup vs baseline: 1.1503x; 1.0163x over previous
import functools

import jax
import jax.numpy as jnp
from jax import lax
from jax.experimental import pallas as pl
from jax.experimental.pallas import tpu as pltpu
from jax.experimental.pallas import tpu_sc as plsc

_E = 320000
_T = 640000
_H = 128
_F = 64


def _silu(v):
    return v * jax.nn.sigmoid(v)


def _bmm(a, w):
    return jax.lax.dot(a.astype(jnp.bfloat16), w.astype(jnp.bfloat16),
                       preferred_element_type=jnp.float32)


_BLK1 = 2560


def _s1_body(x_ref, rbf_ref, Wr1, Wr2, Wkj, bkj, Wji, bji, Wdn, xji_ref, xkj_ref):
    xb = x_ref[...]
    xji_ref[...] = _silu(_bmm(xb, Wji[...]) + bji[...])
    rbf_e = (rbf_ref[...] @ Wr1[...]) @ Wr2[...]
    t = _silu(_bmm(xb, Wkj[...]) + bkj[...]) * rbf_e
    v = _silu(_bmm(t, Wdn[...]))
    xkj_ref[...] = jnp.concatenate([v, jnp.zeros_like(v)], axis=1)


def _stage1(x, rbf, Wr1, Wr2, Wkj, bkj, Wji, bji, Wdn):
    nb = _E // _BLK1
    full = lambda a: pl.BlockSpec(a.shape, lambda i: (0,) * a.ndim)
    return pl.pallas_call(
        _s1_body,
        grid=(nb,),
        in_specs=[
            pl.BlockSpec((_BLK1, _H), lambda i: (i, 0)),
            pl.BlockSpec((_BLK1, 6), lambda i: (i, 0)),
            full(Wr1), full(Wr2), full(Wkj), full(bkj), full(Wji), full(bji),
            full(Wdn),
        ],
        out_specs=[
            pl.BlockSpec((_BLK1, _H), lambda i: (i, 0)),
            pl.BlockSpec((_BLK1, 2 * _F), lambda i: (i, 0)),
        ],
        out_shape=[
            jax.ShapeDtypeStruct((_E, _H), jnp.float32),
            jax.ShapeDtypeStruct((_E, 2 * _F), jnp.float32),
        ],
    )(x, rbf, Wr1, Wr2, Wkj, bkj, Wji, bji, Wdn)


_BLK2 = 2560


def _s2_body(sbf_ref, Ws1, Ws2, out_ref):
    r = (sbf_ref[...] @ Ws1[...]) @ Ws2[...]
    out_ref[...] = jnp.concatenate([r, jnp.zeros_like(r)], axis=1)


def _stage2(sbf, Ws1, Ws2):
    nb = _T // _BLK2
    full = lambda a: pl.BlockSpec(a.shape, lambda i: (0,) * a.ndim)
    return pl.pallas_call(
        _s2_body,
        grid=(nb,),
        in_specs=[
            pl.BlockSpec((_BLK2, 42), lambda i: (i, 0)),
            full(Ws1), full(Ws2),
        ],
        out_specs=pl.BlockSpec((_BLK2, 2 * _F), lambda i: (i, 0)),
        out_shape=jax.ShapeDtypeStruct((_T, 2 * _F), jnp.float32),
    )(sbf, Ws1, Ws2)


_BLK4 = 2560


def _s4_body(seg_ref, xji_ref, x_ref, Wup, Wb1, bb1, Wb2, bb2, Wl, bl,
             Wa1, ba1, Wa2, ba2, Wa3, ba3, Wa4, ba4, out_ref):
    h = xji_ref[...] + _silu(_bmm(seg_ref[...][:, :_F], Wup[...]))
    h = h + _silu(_bmm(_silu(_bmm(h, Wb1[...]) + bb1[...]), Wb2[...]) + bb2[...])
    h = _silu(_bmm(h, Wl[...]) + bl[...]) + x_ref[...]
    h = h + _silu(_bmm(_silu(_bmm(h, Wa1[...]) + ba1[...]), Wa2[...]) + ba2[...])
    h = h + _silu(_bmm(_silu(_bmm(h, Wa3[...]) + ba3[...]), Wa4[...]) + ba4[...])
    out_ref[...] = h


def _stage4(seg, xji, x, Wup, Wb1, bb1, Wb2, bb2, Wl, bl,
            Wa1, ba1, Wa2, ba2, Wa3, ba3, Wa4, ba4):
    nb = _E // _BLK4
    full = lambda a: pl.BlockSpec(a.shape, lambda i: (0,) * a.ndim)
    ws = [Wup, Wb1, bb1, Wb2, bb2, Wl, bl, Wa1, ba1, Wa2, ba2, Wa3, ba3, Wa4, ba4]
    return pl.pallas_call(
        _s4_body,
        grid=(nb,),
        in_specs=[
            pl.BlockSpec((_BLK4, 2 * _F), lambda i: (i, 0)),
            pl.BlockSpec((_BLK4, _H), lambda i: (i, 0)),
            pl.BlockSpec((_BLK4, _H), lambda i: (i, 0)),
        ] + [full(w) for w in ws],
        out_specs=pl.BlockSpec((_BLK4, _H), lambda i: (i, 0)),
        out_shape=jax.ShapeDtypeStruct((_E, _H), jnp.float32),
    )(seg, xji, x, *ws)


def _make_sc_scatter(E, T, F, NB, CH, G, interpret=False, debug_level=9):
    info_nc, info_ns = 2, 16
    RPB = E // NB
    assert RPB * NB == E and RPB % 16 == 0
    PASSES = NB // info_nc
    ACC = RPB + 16
    while max(d for d in range(1, 97) if (ACC // info_ns) % d == 0) < 32:
        ACC += 16
    TSPAN = T // info_ns
    assert TSPAN % CH == 0
    NCH = TSPAN // CH
    assert CH % 16 == 0 and G % 16 == 0
    GB = G + 16
    NV = CH // 16
    ZPT = ACC // info_ns
    zr = 1
    for d in range(2, 97):
        if ZPT % d == 0:
            zr = d
    ZR = zr
    NZ = ZPT // ZR
    CPT = (RPB // info_ns) & ~7
    CPL = RPB - CPT * (info_ns - 1)
    assert CPL % 8 == 0 and CPL >= 0

    mesh = plsc.VectorSubcoreMesh(core_axis_name="c", subcore_axis_name="s",
                                  num_cores=info_nc, num_subcores=info_ns)

    @functools.partial(
        pl.kernel,
        out_type=jax.ShapeDtypeStruct((E, 2 * F), jnp.float32),
        mesh=mesh,
        interpret=interpret,
        compiler_params=pltpu.CompilerParams(needs_layout_passes=False),
        scratch_types=[
            pltpu.VMEM((CH,), jnp.int32),
            pltpu.VMEM((CH,), jnp.int32),
            pltpu.VMEM((GB,), jnp.int32),
            pltpu.VMEM((GB,), jnp.int32),
            pltpu.VMEM((GB,), jnp.int32),
            pltpu.VMEM((GB, 2 * F), jnp.float32),
            pltpu.VMEM((GB, 2 * F), jnp.float32),
            pltpu.VMEM((ZR, 2 * F), jnp.float32),
            pltpu.VMEM_SHARED((ACC, 2 * F), jnp.float32),
            pltpu.SemaphoreType.DMA,
            pltpu.SemaphoreType.DMA,
        ],
    )
    def sc_fn(xkj_hbm, sbf_hbm, idxkj_hbm, idxji_hbm, out_hbm,
              jib, kjb, ckj, ct, cloc,
              xrows, srows, zbuf, acc, sem1, sem2):
        c = lax.axis_index("c")
        s = lax.axis_index("s")
        zero16 = jnp.zeros((16,), jnp.float32)
        iota16 = lax.iota(jnp.int32, 16)
        sent_row = jnp.zeros((16,), jnp.int32) + (RPB + s)
        sent_idx = iota16 + s * 16

        if debug_level >= -1:
            def zb_fill(i, _):
                for q in range(2 * F // 16):
                    zbuf[i, pl.ds(q * 16, 16)] = zero16
                return 0
            lax.fori_loop(0, ZR, zb_fill, 0)

        def flush(cnt):
            def padf(j, _):
                o = jnp.minimum(cnt + j * 16, G)
                ckj[pl.ds(o, 16)] = sent_idx
                ct[pl.ds(o, 16)] = sent_idx
                cloc[pl.ds(o, 16)] = sent_row
                return 0
            lax.fori_loop(0, G // 16 + 1, padf, 0)

            d1 = pltpu.async_copy(xkj_hbm.at[ckj], xrows, sem1)
            d2 = pltpu.async_copy(sbf_hbm.at[ct], srows, sem2)
            d1.wait()
            d2.wait()

            def mult(r, _):
                for q in range(F // 16):
                    xrows[r, pl.ds(q * 16, 16)] = (
                        xrows[r, pl.ds(q * 16, 16)]
                        * srows[r, pl.ds(q * 16, 16)])
                return 0
            lax.fori_loop(0, GB, mult, 0)
            pltpu.sync_copy(xrows, acc.at[cloc], add=True)

        def one_pass(p, _):
            b = c * PASSES + p
            lo = b * RPB
            if debug_level >= 0:
                for z in range(NZ):
                    pltpu.sync_copy(zbuf, acc.at[pl.ds(s * ZPT + z * ZR, ZR)])
            plsc.subcore_barrier()

            def one_chunk(ch, cnt):
                off = s * TSPAN + ch * CH
                pltpu.sync_copy(idxji_hbm.at[pl.ds(off, CH)], jib)
                pltpu.sync_copy(idxkj_hbm.at[pl.ds(off, CH)], kjb)
                if debug_level < 3:
                    return cnt

                def compact(v, cnt):
                    vji = jib[pl.ds(v * 16, 16)]
                    vkj = kjb[pl.ds(v * 16, 16)]
                    msk = (vji >= lo) & (vji < lo + RPB)
                    loc = vji - lo
                    tv = off + v * 16 + iota16
                    plsc.store_compressed(ckj.at[pl.ds(cnt, 16)], vkj, mask=msk)
                    plsc.store_compressed(ct.at[pl.ds(cnt, 16)], tv, mask=msk)
                    plsc.store_compressed(cloc.at[pl.ds(cnt, 16)], loc, mask=msk)
                    cnt = cnt + plsc.all_reduce_population_count(msk)[0]

                    def do_flush(cc):
                        if debug_level >= 4:
                            flush(cc)
                        return jnp.int32(0)
                    return lax.cond(cnt >= G - 15, do_flush,
                                    lambda cc: cc, cnt)
                return lax.fori_loop(0, NV, compact, cnt)
            if debug_level >= 2:
                cnt = lax.fori_loop(0, NCH, one_chunk, jnp.int32(0))
                if debug_level >= 4:
                    flush(cnt)

            plsc.subcore_barrier()
            if debug_level >= 1:
                @pl.when(s < info_ns - 1)
                def _():
                    pltpu.sync_copy(acc.at[pl.ds(s * CPT, CPT)],
                                    out_hbm.at[pl.ds(b * RPB + s * CPT, CPT)])

                @pl.when(s == info_ns - 1)
                def _():
                    pltpu.sync_copy(
                        acc.at[pl.ds((info_ns - 1) * CPT, CPL)],
                        out_hbm.at[pl.ds(b * RPB + (info_ns - 1) * CPT, CPL)])
            plsc.subcore_barrier()
            return 0
        lax.fori_loop(0, PASSES, one_pass, 0)

    return sc_fn


_sc_scatter = None
_DBG = 9


def _get_sc_scatter():
    global _sc_scatter
    if _sc_scatter is None:
        _sc_scatter = _make_sc_scatter(_E, _T, _F, NB=32, CH=4000, G=112,
                                       debug_level=_DBG)
    return _sc_scatter


def kernel(x, rbf, sbf, idx_kj, idx_ji, W_rbf1, W_rbf2, W_sbf1, W_sbf2,
           W_kj, b_kj, W_ji, b_ji, W_down, W_up,
           W_bs1, b_bs1, W_bs2, b_bs2, W_lin, b_lin,
           W_as1, b_as1, W_as2, b_as2, W_as3, b_as3, W_as4, b_as4):
    r2 = lambda v: v.reshape(1, -1)
    xji, xkj = _stage1(x, rbf, W_rbf1, W_rbf2, W_kj, r2(b_kj), W_ji,
                       r2(b_ji), W_down)
    sbf_e = _stage2(sbf, W_sbf1, W_sbf2)
    seg = _get_sc_scatter()(xkj, sbf_e, idx_kj.astype(jnp.int32),
                            idx_ji.astype(jnp.int32))
    if _DBG < 9:
        fix = jax.ops.segment_sum(
            xkj[idx_kj, :_F] * sbf_e[:, :_F], idx_ji, num_segments=_E)
        seg = seg.at[:, :_F].add(fix)
    return _stage4(seg, xji, x, W_up, W_bs1, r2(b_bs1), W_bs2, r2(b_bs2),
                   W_lin, r2(b_lin), W_as1, r2(b_as1), W_as2, r2(b_as2),
                   W_as3, r2(b_as3), W_as4, r2(b_as4))


# speedup vs baseline: 1.1508x; 1.0004x over previous
import functools

import jax
import jax.numpy as jnp
from jax import lax
from jax.experimental import pallas as pl
from jax.experimental.pallas import tpu as pltpu
from jax.experimental.pallas import tpu_sc as plsc

_E = 320000
_T = 640000
_H = 128
_F = 64


def _silu(v):
    return v * jax.nn.sigmoid(v)


def _bmm(a, w):
    return jax.lax.dot(a.astype(jnp.bfloat16), w.astype(jnp.bfloat16),
                       preferred_element_type=jnp.float32)


_BLK1 = 2560


def _s1_body(x_ref, rbf_ref, Wr1, Wr2, Wkj, bkj, Wji, bji, Wdn, xji_ref, xkj_ref):
    xb = x_ref[...]
    xji_ref[...] = _silu(_bmm(xb, Wji[...]) + bji[...])
    rbf_e = (rbf_ref[...] @ Wr1[...]) @ Wr2[...]
    t = _silu(_bmm(xb, Wkj[...]) + bkj[...]) * rbf_e
    v = _silu(_bmm(t, Wdn[...]))
    xkj_ref[...] = jnp.concatenate([v, jnp.zeros_like(v)], axis=1)


def _stage1(x, rbf, Wr1, Wr2, Wkj, bkj, Wji, bji, Wdn):
    nb = _E // _BLK1
    full = lambda a: pl.BlockSpec(a.shape, lambda i: (0,) * a.ndim)
    return pl.pallas_call(
        _s1_body,
        grid=(nb,),
        in_specs=[
            pl.BlockSpec((_BLK1, _H), lambda i: (i, 0)),
            pl.BlockSpec((_BLK1, 6), lambda i: (i, 0)),
            full(Wr1), full(Wr2), full(Wkj), full(bkj), full(Wji), full(bji),
            full(Wdn),
        ],
        out_specs=[
            pl.BlockSpec((_BLK1, _H), lambda i: (i, 0)),
            pl.BlockSpec((_BLK1, 2 * _F), lambda i: (i, 0)),
        ],
        out_shape=[
            jax.ShapeDtypeStruct((_E, _H), jnp.float32),
            jax.ShapeDtypeStruct((_E, 2 * _F), jnp.float32),
        ],
    )(x, rbf, Wr1, Wr2, Wkj, bkj, Wji, bji, Wdn)


_BLK2 = 2560


def _s2_body(sbf_ref, Ws1, Ws2, out_ref):
    r = (sbf_ref[...] @ Ws1[...]) @ Ws2[...]
    out_ref[...] = jnp.concatenate([r, jnp.zeros_like(r)], axis=1)


def _stage2(sbf, Ws1, Ws2):
    nb = _T // _BLK2
    full = lambda a: pl.BlockSpec(a.shape, lambda i: (0,) * a.ndim)
    return pl.pallas_call(
        _s2_body,
        grid=(nb,),
        in_specs=[
            pl.BlockSpec((_BLK2, 42), lambda i: (i, 0)),
            full(Ws1), full(Ws2),
        ],
        out_specs=pl.BlockSpec((_BLK2, 2 * _F), lambda i: (i, 0)),
        out_shape=jax.ShapeDtypeStruct((_T, 2 * _F), jnp.float32),
    )(sbf, Ws1, Ws2)


_BLK4 = 2560


def _s4_body(seg_ref, xji_ref, x_ref, Wup, Wb1, bb1, Wb2, bb2, Wl, bl,
             Wa1, ba1, Wa2, ba2, Wa3, ba3, Wa4, ba4, out_ref):
    h = xji_ref[...] + _silu(_bmm(seg_ref[...][:, :_F], Wup[...]))
    h = h + _silu(_bmm(_silu(_bmm(h, Wb1[...]) + bb1[...]), Wb2[...]) + bb2[...])
    h = _silu(_bmm(h, Wl[...]) + bl[...]) + x_ref[...]
    h = h + _silu(_bmm(_silu(_bmm(h, Wa1[...]) + ba1[...]), Wa2[...]) + ba2[...])
    h = h + _silu(_bmm(_silu(_bmm(h, Wa3[...]) + ba3[...]), Wa4[...]) + ba4[...])
    out_ref[...] = h


def _stage4(seg, xji, x, Wup, Wb1, bb1, Wb2, bb2, Wl, bl,
            Wa1, ba1, Wa2, ba2, Wa3, ba3, Wa4, ba4):
    nb = _E // _BLK4
    full = lambda a: pl.BlockSpec(a.shape, lambda i: (0,) * a.ndim)
    ws = [Wup, Wb1, bb1, Wb2, bb2, Wl, bl, Wa1, ba1, Wa2, ba2, Wa3, ba3, Wa4, ba4]
    return pl.pallas_call(
        _s4_body,
        grid=(nb,),
        in_specs=[
            pl.BlockSpec((_BLK4, 2 * _F), lambda i: (i, 0)),
            pl.BlockSpec((_BLK4, _H), lambda i: (i, 0)),
            pl.BlockSpec((_BLK4, _H), lambda i: (i, 0)),
        ] + [full(w) for w in ws],
        out_specs=pl.BlockSpec((_BLK4, _H), lambda i: (i, 0)),
        out_shape=jax.ShapeDtypeStruct((_E, _H), jnp.float32),
    )(seg, xji, x, *ws)


def _make_sc_scatter(E, T, F, NB, CH, G, interpret=False):
    info_nc, info_ns = 2, 16
    RPB = E // NB
    assert RPB * NB == E and RPB % 16 == 0
    PASSES = NB // info_nc
    ACC = RPB + 16
    while max(d for d in range(1, 97) if (ACC // info_ns) % d == 0) < 32:
        ACC += 16
    TSPAN = T // info_ns
    assert TSPAN % CH == 0
    NCH = TSPAN // CH
    assert CH % 16 == 0 and G % 16 == 0
    GB = G + 16
    NV = CH // 16
    ZPT = ACC // info_ns
    zr = 1
    for d in range(2, 97):
        if ZPT % d == 0:
            zr = d
    ZR = zr
    NZ = ZPT // ZR
    CPT = (RPB // info_ns) & ~7
    CPL = RPB - CPT * (info_ns - 1)
    assert CPL % 8 == 0 and CPL >= 0

    mesh = plsc.VectorSubcoreMesh(core_axis_name="c", subcore_axis_name="s",
                                  num_cores=info_nc, num_subcores=info_ns)

    @functools.partial(
        pl.kernel,
        out_type=jax.ShapeDtypeStruct((E, 2 * F), jnp.float32),
        mesh=mesh,
        interpret=interpret,
        compiler_params=pltpu.CompilerParams(needs_layout_passes=False),
        scratch_types=[
            pltpu.VMEM((CH,), jnp.int32),
            pltpu.VMEM((CH,), jnp.int32),
            pltpu.VMEM((GB,), jnp.int32),
            pltpu.VMEM((GB,), jnp.int32),
            pltpu.VMEM((GB,), jnp.int32),
            pltpu.VMEM((GB, 2 * F), jnp.float32),
            pltpu.VMEM((GB, 2 * F), jnp.float32),
            pltpu.VMEM((ZR, 2 * F), jnp.float32),
            pltpu.VMEM_SHARED((ACC, 2 * F), jnp.float32),
            pltpu.SemaphoreType.DMA,
            pltpu.SemaphoreType.DMA,
        ],
    )
    def sc_fn(xkj_hbm, sbf_hbm, idxkj_hbm, idxji_hbm, out_hbm,
              jib, kjb, ckj, ct, cloc,
              xrows, srows, zbuf, acc, sem1, sem2):
        c = lax.axis_index("c")
        s = lax.axis_index("s")
        zero16 = jnp.zeros((16,), jnp.float32)
        iota16 = lax.iota(jnp.int32, 16)
        sent_row = jnp.zeros((16,), jnp.int32) + (RPB + s)
        sent_idx = iota16 + s * 16

        def zb_fill(i, _):
            for q in range(2 * F // 16):
                zbuf[i, pl.ds(q * 16, 16)] = zero16
            return 0
        lax.fori_loop(0, ZR, zb_fill, 0)

        def flush(cnt):
            def padf(j, _):
                o = jnp.minimum(cnt + j * 16, G)
                ckj[pl.ds(o, 16)] = sent_idx
                ct[pl.ds(o, 16)] = sent_idx
                cloc[pl.ds(o, 16)] = sent_row
                return 0
            lax.fori_loop(0, G // 16 + 1, padf, 0)

            d1 = pltpu.async_copy(xkj_hbm.at[ckj], xrows, sem1)
            d2 = pltpu.async_copy(sbf_hbm.at[ct], srows, sem2)
            d1.wait()
            d2.wait()

            def mult(r, _):
                for q in range(F // 16):
                    xrows[r, pl.ds(q * 16, 16)] = (
                        xrows[r, pl.ds(q * 16, 16)]
                        * srows[r, pl.ds(q * 16, 16)])
                return 0
            lax.fori_loop(0, GB, mult, 0)
            pltpu.sync_copy(xrows, acc.at[cloc], add=True)

        def one_pass(p, _):
            b = c * PASSES + p
            lo = b * RPB
            for z in range(NZ):
                pltpu.sync_copy(zbuf, acc.at[pl.ds(s * ZPT + z * ZR, ZR)])
            plsc.subcore_barrier()

            def one_chunk(ch, cnt):
                off = s * TSPAN + ch * CH
                pltpu.sync_copy(idxji_hbm.at[pl.ds(off, CH)], jib)
                pltpu.sync_copy(idxkj_hbm.at[pl.ds(off, CH)], kjb)

                def compact(v, cnt):
                    vji = jib[pl.ds(v * 16, 16)]
                    vkj = kjb[pl.ds(v * 16, 16)]
                    msk = (vji >= lo) & (vji < lo + RPB)
                    loc = vji - lo
                    tv = off + v * 16 + iota16
                    plsc.store_compressed(ckj.at[pl.ds(cnt, 16)], vkj, mask=msk)
                    plsc.store_compressed(ct.at[pl.ds(cnt, 16)], tv, mask=msk)
                    plsc.store_compressed(cloc.at[pl.ds(cnt, 16)], loc, mask=msk)
                    cnt = cnt + plsc.all_reduce_population_count(msk)[0]

                    def do_flush(cc):
                        flush(cc)
                        return jnp.int32(0)
                    return lax.cond(cnt >= G - 15, do_flush,
                                    lambda cc: cc, cnt)
                return lax.fori_loop(0, NV, compact, cnt)
            cnt = lax.fori_loop(0, NCH, one_chunk, jnp.int32(0))
            flush(cnt)

            plsc.subcore_barrier()
            @pl.when(s < info_ns - 1)
            def _():
                pltpu.sync_copy(acc.at[pl.ds(s * CPT, CPT)],
                                out_hbm.at[pl.ds(b * RPB + s * CPT, CPT)])

            @pl.when(s == info_ns - 1)
            def _():
                pltpu.sync_copy(
                    acc.at[pl.ds((info_ns - 1) * CPT, CPL)],
                    out_hbm.at[pl.ds(b * RPB + (info_ns - 1) * CPT, CPL)])
            plsc.subcore_barrier()
            return 0
        lax.fori_loop(0, PASSES, one_pass, 0)

    return sc_fn


_sc_scatter = None


def _get_sc_scatter():
    global _sc_scatter
    if _sc_scatter is None:
        _sc_scatter = _make_sc_scatter(_E, _T, _F, NB=32, CH=4000, G=112)
    return _sc_scatter


def kernel(x, rbf, sbf, idx_kj, idx_ji, W_rbf1, W_rbf2, W_sbf1, W_sbf2,
           W_kj, b_kj, W_ji, b_ji, W_down, W_up,
           W_bs1, b_bs1, W_bs2, b_bs2, W_lin, b_lin,
           W_as1, b_as1, W_as2, b_as2, W_as3, b_as3, W_as4, b_as4):
    r2 = lambda v: v.reshape(1, -1)
    xji, xkj = _stage1(x, rbf, W_rbf1, W_rbf2, W_kj, r2(b_kj), W_ji,
                       r2(b_ji), W_down)
    sbf_e = _stage2(sbf, W_sbf1, W_sbf2)
    seg = _get_sc_scatter()(xkj, sbf_e, idx_kj.astype(jnp.int32),
                            idx_ji.astype(jnp.int32))
    return _stage4(seg, xji, x, W_up, W_bs1, r2(b_bs1), W_bs2, r2(b_bs2),
                   W_lin, r2(b_lin), W_as1, r2(b_as1), W_as2, r2(b_as2),
                   W_as3, r2(b_as3), W_as4, r2(b_as4))


# speedup vs baseline: 1.2366x; 1.0746x over previous
import functools

import jax
import jax.numpy as jnp
from jax import lax
from jax.experimental import pallas as pl
from jax.experimental.pallas import tpu as pltpu
from jax.experimental.pallas import tpu_sc as plsc

_E = 320000
_T = 640000
_H = 128
_F = 64


def _silu(v):
    return v * jax.nn.sigmoid(v)


def _bmm(a, w):
    return jax.lax.dot(a.astype(jnp.bfloat16), w.astype(jnp.bfloat16),
                       preferred_element_type=jnp.float32)


_BLK1 = 2560


def _s1_body(x_ref, rbf_ref, Wr1, Wr2, Wkj, bkj, Wji, bji, Wdn, xji_ref, xkj_ref):
    xb = x_ref[...]
    xji_ref[...] = _silu(_bmm(xb, Wji[...]) + bji[...])
    rbf_e = (rbf_ref[...] @ Wr1[...]) @ Wr2[...]
    t = _silu(_bmm(xb, Wkj[...]) + bkj[...]) * rbf_e
    v = _silu(_bmm(t, Wdn[...]))
    xkj_ref[...] = jnp.concatenate([v, jnp.zeros_like(v)], axis=1)


def _stage1(x, rbf, Wr1, Wr2, Wkj, bkj, Wji, bji, Wdn):
    nb = _E // _BLK1
    full = lambda a: pl.BlockSpec(a.shape, lambda i: (0,) * a.ndim)
    return pl.pallas_call(
        _s1_body,
        grid=(nb,),
        in_specs=[
            pl.BlockSpec((_BLK1, _H), lambda i: (i, 0)),
            pl.BlockSpec((_BLK1, 6), lambda i: (i, 0)),
            full(Wr1), full(Wr2), full(Wkj), full(bkj), full(Wji), full(bji),
            full(Wdn),
        ],
        out_specs=[
            pl.BlockSpec((_BLK1, _H), lambda i: (i, 0)),
            pl.BlockSpec((_BLK1, 2 * _F), lambda i: (i, 0)),
        ],
        out_shape=[
            jax.ShapeDtypeStruct((_E, _H), jnp.float32),
            jax.ShapeDtypeStruct((_E, 2 * _F), jnp.float32),
        ],
    )(x, rbf, Wr1, Wr2, Wkj, bkj, Wji, bji, Wdn)


_BLK2 = 2560


def _s2_body(sbf_ref, Ws1, Ws2, out_ref):
    r = (sbf_ref[...] @ Ws1[...]) @ Ws2[...]
    out_ref[...] = jnp.concatenate([r, jnp.zeros_like(r)], axis=1)


def _stage2(sbf, Ws1, Ws2):
    nb = _T // _BLK2
    full = lambda a: pl.BlockSpec(a.shape, lambda i: (0,) * a.ndim)
    return pl.pallas_call(
        _s2_body,
        grid=(nb,),
        in_specs=[
            pl.BlockSpec((_BLK2, 42), lambda i: (i, 0)),
            full(Ws1), full(Ws2),
        ],
        out_specs=pl.BlockSpec((_BLK2, 2 * _F), lambda i: (i, 0)),
        out_shape=jax.ShapeDtypeStruct((_T, 2 * _F), jnp.float32),
    )(sbf, Ws1, Ws2)


_BLK4 = 2560


def _s4_body(seg_ref, xji_ref, x_ref, Wup, Wb1, bb1, Wb2, bb2, Wl, bl,
             Wa1, ba1, Wa2, ba2, Wa3, ba3, Wa4, ba4, out_ref):
    h = xji_ref[...] + _silu(_bmm(seg_ref[...][:, :_F], Wup[...]))
    h = h + _silu(_bmm(_silu(_bmm(h, Wb1[...]) + bb1[...]), Wb2[...]) + bb2[...])
    h = _silu(_bmm(h, Wl[...]) + bl[...]) + x_ref[...]
    h = h + _silu(_bmm(_silu(_bmm(h, Wa1[...]) + ba1[...]), Wa2[...]) + ba2[...])
    h = h + _silu(_bmm(_silu(_bmm(h, Wa3[...]) + ba3[...]), Wa4[...]) + ba4[...])
    out_ref[...] = h


def _stage4(seg, xji, x, Wup, Wb1, bb1, Wb2, bb2, Wl, bl,
            Wa1, ba1, Wa2, ba2, Wa3, ba3, Wa4, ba4):
    nb = _E // _BLK4
    full = lambda a: pl.BlockSpec(a.shape, lambda i: (0,) * a.ndim)
    ws = [Wup, Wb1, bb1, Wb2, bb2, Wl, bl, Wa1, ba1, Wa2, ba2, Wa3, ba3, Wa4, ba4]
    return pl.pallas_call(
        _s4_body,
        grid=(nb,),
        in_specs=[
            pl.BlockSpec((_BLK4, 2 * _F), lambda i: (i, 0)),
            pl.BlockSpec((_BLK4, _H), lambda i: (i, 0)),
            pl.BlockSpec((_BLK4, _H), lambda i: (i, 0)),
        ] + [full(w) for w in ws],
        out_specs=pl.BlockSpec((_BLK4, _H), lambda i: (i, 0)),
        out_shape=jax.ShapeDtypeStruct((_E, _H), jnp.float32),
    )(seg, xji, x, *ws)


def _make_sc_scatter(E, T, F, NB, CH, G, interpret=False):
    info_nc, info_ns = 2, 16
    RPB = E // NB
    assert RPB * NB == E and RPB % 16 == 0
    PASSES = NB // info_nc
    ACC = RPB + 16
    while max(d for d in range(1, 97) if (ACC // info_ns) % d == 0) < 32:
        ACC += 16
    TSPAN = T // info_ns
    assert TSPAN % CH == 0
    NCH = TSPAN // CH
    assert CH % 16 == 0 and G % 16 == 0
    GB = G + 16
    NV = CH // 16
    ZPT = ACC // info_ns
    zr = 1
    for d in range(2, 97):
        if ZPT % d == 0:
            zr = d
    ZR = zr
    NZ = ZPT // ZR
    CPT = (RPB // info_ns) & ~7
    CPL = RPB - CPT * (info_ns - 1)
    assert CPL % 8 == 0 and CPL >= 0

    mesh = plsc.VectorSubcoreMesh(core_axis_name="c", subcore_axis_name="s",
                                  num_cores=info_nc, num_subcores=info_ns)

    @functools.partial(
        pl.kernel,
        out_type=jax.ShapeDtypeStruct((E, 2 * F), jnp.float32),
        mesh=mesh,
        interpret=interpret,
        compiler_params=pltpu.CompilerParams(needs_layout_passes=False),
        scratch_types=[
            pltpu.VMEM((CH,), jnp.int32),
            pltpu.VMEM((CH,), jnp.int32),
            pltpu.VMEM((CH,), jnp.int32),
            pltpu.VMEM((CH,), jnp.int32),
            pltpu.VMEM((GB,), jnp.int32),
            pltpu.VMEM((GB,), jnp.int32),
            pltpu.VMEM((GB,), jnp.int32),
            pltpu.VMEM((GB, 2 * F), jnp.float32),
            pltpu.VMEM((GB, 2 * F), jnp.float32),
            pltpu.VMEM((ZR, 2 * F), jnp.float32),
            pltpu.VMEM_SHARED((ACC, 2 * F), jnp.float32),
            pltpu.SemaphoreType.DMA,
            pltpu.SemaphoreType.DMA,
            pltpu.SemaphoreType.DMA,
            pltpu.SemaphoreType.DMA,
        ],
    )
    def sc_fn(xkj_hbm, sbf_hbm, idxkj_hbm, idxji_hbm, out_hbm,
              jib0, kjb0, jib1, kjb1, ckj, ct, cloc,
              xrows, srows, zbuf, acc, sem1, sem2, sem3, sem4):
        c = lax.axis_index("c")
        s = lax.axis_index("s")
        zero16 = jnp.zeros((16,), jnp.float32)
        iota16 = lax.iota(jnp.int32, 16)
        sent_row = jnp.zeros((16,), jnp.int32) + (RPB + s)
        sent_idx = iota16 + s * 16

        def zb_fill(i, _):
            for q in range(2 * F // 16):
                zbuf[i, pl.ds(q * 16, 16)] = zero16
            return 0
        lax.fori_loop(0, ZR, zb_fill, 0)

        def flush(cnt):
            def padf(j, _):
                o = jnp.minimum(cnt + j * 16, G)
                ckj[pl.ds(o, 16)] = sent_idx
                ct[pl.ds(o, 16)] = sent_idx
                cloc[pl.ds(o, 16)] = sent_row
                return 0
            lax.fori_loop(0, G // 16 + 1, padf, 0)

            d1 = pltpu.async_copy(xkj_hbm.at[ckj], xrows, sem1)
            d2 = pltpu.async_copy(sbf_hbm.at[ct], srows, sem2)
            d1.wait()
            d2.wait()

            def mult(r, _):
                for q in range(F // 16):
                    xrows[r, pl.ds(q * 16, 16)] = (
                        xrows[r, pl.ds(q * 16, 16)]
                        * srows[r, pl.ds(q * 16, 16)])
                return 0
            lax.fori_loop(0, GB, mult, 0)
            pltpu.sync_copy(xrows, acc.at[cloc], add=True)

        def one_pass(p, _):
            b = c * PASSES + p
            lo = b * RPB
            for z in range(NZ):
                pltpu.sync_copy(zbuf, acc.at[pl.ds(s * ZPT + z * ZR, ZR)])
            plsc.subcore_barrier()

            bufs = ((jib0, kjb0, sem3), (jib1, kjb1, sem4))

            def chunk_fetch(ch, jib, kjb, semj):
                off = s * TSPAN + ch * CH
                dj = pltpu.async_copy(idxji_hbm.at[pl.ds(off, CH)], jib, semj)
                dk = pltpu.async_copy(idxkj_hbm.at[pl.ds(off, CH)], kjb, semj)
                return dj, dk

            def chunk_scan(ch, jib, kjb, cnt):
                off = s * TSPAN + ch * CH

                def compact(v, cnt):
                    vji = jib[pl.ds(v * 16, 16)]
                    vkj = kjb[pl.ds(v * 16, 16)]
                    msk = (vji >= lo) & (vji < lo + RPB)
                    loc = vji - lo
                    tv = off + v * 16 + iota16
                    plsc.store_compressed(ckj.at[pl.ds(cnt, 16)], vkj, mask=msk)
                    plsc.store_compressed(ct.at[pl.ds(cnt, 16)], tv, mask=msk)
                    plsc.store_compressed(cloc.at[pl.ds(cnt, 16)], loc, mask=msk)
                    cnt = cnt + plsc.all_reduce_population_count(msk)[0]

                    def do_flush(cc):
                        flush(cc)
                        return jnp.int32(0)
                    return lax.cond(cnt >= G - 15, do_flush,
                                    lambda cc: cc, cnt)
                return lax.fori_loop(0, NV, compact, cnt)

            cnt = jnp.int32(0)
            pend = chunk_fetch(0, *bufs[0])
            for ch in range(NCH):
                jib, kjb, _ = bufs[ch % 2]
                for d in pend:
                    d.wait()
                if ch + 1 < NCH:
                    pend = chunk_fetch(ch + 1, *bufs[(ch + 1) % 2])
                cnt = chunk_scan(ch, jib, kjb, cnt)
            flush(cnt)

            plsc.subcore_barrier()
            @pl.when(s < info_ns - 1)
            def _():
                pltpu.sync_copy(acc.at[pl.ds(s * CPT, CPT)],
                                out_hbm.at[pl.ds(b * RPB + s * CPT, CPT)])

            @pl.when(s == info_ns - 1)
            def _():
                pltpu.sync_copy(
                    acc.at[pl.ds((info_ns - 1) * CPT, CPL)],
                    out_hbm.at[pl.ds(b * RPB + (info_ns - 1) * CPT, CPL)])
            plsc.subcore_barrier()
            return 0
        lax.fori_loop(0, PASSES, one_pass, 0)

    return sc_fn


_sc_scatter = None


def _get_sc_scatter():
    global _sc_scatter
    if _sc_scatter is None:
        _sc_scatter = _make_sc_scatter(_E, _T, _F, NB=32, CH=2000, G=112)
    return _sc_scatter


def kernel(x, rbf, sbf, idx_kj, idx_ji, W_rbf1, W_rbf2, W_sbf1, W_sbf2,
           W_kj, b_kj, W_ji, b_ji, W_down, W_up,
           W_bs1, b_bs1, W_bs2, b_bs2, W_lin, b_lin,
           W_as1, b_as1, W_as2, b_as2, W_as3, b_as3, W_as4, b_as4):
    r2 = lambda v: v.reshape(1, -1)
    xji, xkj = _stage1(x, rbf, W_rbf1, W_rbf2, W_kj, r2(b_kj), W_ji,
                       r2(b_ji), W_down)
    sbf_e = _stage2(sbf, W_sbf1, W_sbf2)
    seg = _get_sc_scatter()(xkj, sbf_e, idx_kj.astype(jnp.int32),
                            idx_ji.astype(jnp.int32))
    return _stage4(seg, xji, x, W_up, W_bs1, r2(b_bs1), W_bs2, r2(b_bs2),
                   W_lin, r2(b_lin), W_as1, r2(b_as1), W_as2, r2(b_as2),
                   W_as3, r2(b_as3), W_as4, r2(b_as4))


# speedup vs baseline: 1.2389x; 1.0018x over previous
import functools

import jax
import jax.numpy as jnp
from jax import lax
from jax.experimental import pallas as pl
from jax.experimental.pallas import tpu as pltpu
from jax.experimental.pallas import tpu_sc as plsc

_E = 320000
_T = 640000
_H = 128
_F = 64


def _silu(v):
    return v * jax.nn.sigmoid(v)


def _bmm(a, w):
    return jax.lax.dot(a.astype(jnp.bfloat16), w.astype(jnp.bfloat16),
                       preferred_element_type=jnp.float32)


_BLK1 = 2560


def _s1_body(x_ref, rbf_ref, Wr1, Wr2, Wkj, bkj, Wji, bji, Wdn, xji_ref, xkj_ref):
    xb = x_ref[...]
    xji_ref[...] = _silu(_bmm(xb, Wji[...]) + bji[...])
    rbf_e = (rbf_ref[...] @ Wr1[...]) @ Wr2[...]
    t = _silu(_bmm(xb, Wkj[...]) + bkj[...]) * rbf_e
    v = _silu(_bmm(t, Wdn[...]))
    xkj_ref[...] = jnp.concatenate([v, jnp.zeros_like(v)], axis=1)


def _stage1(x, rbf, Wr1, Wr2, Wkj, bkj, Wji, bji, Wdn):
    nb = _E // _BLK1
    full = lambda a: pl.BlockSpec(a.shape, lambda i: (0,) * a.ndim)
    return pl.pallas_call(
        _s1_body,
        grid=(nb,),
        in_specs=[
            pl.BlockSpec((_BLK1, _H), lambda i: (i, 0)),
            pl.BlockSpec((_BLK1, 6), lambda i: (i, 0)),
            full(Wr1), full(Wr2), full(Wkj), full(bkj), full(Wji), full(bji),
            full(Wdn),
        ],
        out_specs=[
            pl.BlockSpec((_BLK1, _H), lambda i: (i, 0)),
            pl.BlockSpec((_BLK1, 2 * _F), lambda i: (i, 0)),
        ],
        out_shape=[
            jax.ShapeDtypeStruct((_E, _H), jnp.float32),
            jax.ShapeDtypeStruct((_E, 2 * _F), jnp.float32),
        ],
    )(x, rbf, Wr1, Wr2, Wkj, bkj, Wji, bji, Wdn)


_BLK2 = 2560


def _s2_body(sbf_ref, Ws1, Ws2, out_ref):
    r = (sbf_ref[...] @ Ws1[...]) @ Ws2[...]
    out_ref[...] = jnp.concatenate([r, jnp.zeros_like(r)], axis=1)


def _stage2(sbf, Ws1, Ws2):
    nb = _T // _BLK2
    full = lambda a: pl.BlockSpec(a.shape, lambda i: (0,) * a.ndim)
    return pl.pallas_call(
        _s2_body,
        grid=(nb,),
        in_specs=[
            pl.BlockSpec((_BLK2, 42), lambda i: (i, 0)),
            full(Ws1), full(Ws2),
        ],
        out_specs=pl.BlockSpec((_BLK2, 2 * _F), lambda i: (i, 0)),
        out_shape=jax.ShapeDtypeStruct((_T, 2 * _F), jnp.float32),
    )(sbf, Ws1, Ws2)


_BLK4 = 2560


def _s4_body(seg_ref, xji_ref, x_ref, Wup, Wb1, bb1, Wb2, bb2, Wl, bl,
             Wa1, ba1, Wa2, ba2, Wa3, ba3, Wa4, ba4, out_ref):
    h = xji_ref[...] + _silu(_bmm(seg_ref[...][:, :_F], Wup[...]))
    h = h + _silu(_bmm(_silu(_bmm(h, Wb1[...]) + bb1[...]), Wb2[...]) + bb2[...])
    h = _silu(_bmm(h, Wl[...]) + bl[...]) + x_ref[...]
    h = h + _silu(_bmm(_silu(_bmm(h, Wa1[...]) + ba1[...]), Wa2[...]) + ba2[...])
    h = h + _silu(_bmm(_silu(_bmm(h, Wa3[...]) + ba3[...]), Wa4[...]) + ba4[...])
    out_ref[...] = h


def _stage4(seg, xji, x, Wup, Wb1, bb1, Wb2, bb2, Wl, bl,
            Wa1, ba1, Wa2, ba2, Wa3, ba3, Wa4, ba4):
    nb = _E // _BLK4
    full = lambda a: pl.BlockSpec(a.shape, lambda i: (0,) * a.ndim)
    ws = [Wup, Wb1, bb1, Wb2, bb2, Wl, bl, Wa1, ba1, Wa2, ba2, Wa3, ba3, Wa4, ba4]
    return pl.pallas_call(
        _s4_body,
        grid=(nb,),
        in_specs=[
            pl.BlockSpec((_BLK4, 2 * _F), lambda i: (i, 0)),
            pl.BlockSpec((_BLK4, _H), lambda i: (i, 0)),
            pl.BlockSpec((_BLK4, _H), lambda i: (i, 0)),
        ] + [full(w) for w in ws],
        out_specs=pl.BlockSpec((_BLK4, _H), lambda i: (i, 0)),
        out_shape=jax.ShapeDtypeStruct((_E, _H), jnp.float32),
    )(seg, xji, x, *ws)


def _make_sc_scatter(E, T, F, NB, CH, G, interpret=False):
    info_nc, info_ns = 2, 16
    RPB = E // NB
    assert RPB * NB == E and RPB % 16 == 0
    PASSES = NB // info_nc
    ACC = RPB + 16
    while max(d for d in range(1, 97) if (ACC // info_ns) % d == 0) < 32:
        ACC += 16
    TSPAN = T // info_ns
    assert TSPAN % CH == 0
    NCH = TSPAN // CH
    assert CH % 16 == 0 and G % 16 == 0
    GB = G + 16
    NV = CH // 16
    ZPT = ACC // info_ns
    zr = 1
    for d in range(2, 97):
        if ZPT % d == 0:
            zr = d
    ZR = zr
    NZ = ZPT // ZR
    CPT = (RPB // info_ns) & ~7
    CPL = RPB - CPT * (info_ns - 1)
    assert CPL % 8 == 0 and CPL >= 0

    mesh = plsc.VectorSubcoreMesh(core_axis_name="c", subcore_axis_name="s",
                                  num_cores=info_nc, num_subcores=info_ns)

    @functools.partial(
        pl.kernel,
        out_type=jax.ShapeDtypeStruct((E, 2 * F), jnp.float32),
        mesh=mesh,
        interpret=interpret,
        compiler_params=pltpu.CompilerParams(needs_layout_passes=False),
        scratch_types=[
            pltpu.VMEM((CH,), jnp.int32),
            pltpu.VMEM((CH,), jnp.int32),
            pltpu.VMEM((CH,), jnp.int32),
            pltpu.VMEM((CH,), jnp.int32),
            pltpu.VMEM((GB,), jnp.int32),
            pltpu.VMEM((GB,), jnp.int32),
            pltpu.VMEM((GB,), jnp.int32),
            pltpu.VMEM((GB, 2 * F), jnp.float32),
            pltpu.VMEM((GB, 2 * F), jnp.float32),
            pltpu.VMEM((ZR, 2 * F), jnp.float32),
            pltpu.VMEM_SHARED((ACC, 2 * F), jnp.float32),
            pltpu.SemaphoreType.DMA,
            pltpu.SemaphoreType.DMA,
            pltpu.SemaphoreType.DMA,
            pltpu.SemaphoreType.DMA,
        ],
    )
    def sc_fn(xkj_hbm, sbf_hbm, idxkj_hbm, idxji_hbm, out_hbm,
              jib0, kjb0, jib1, kjb1, ckj, ct, cloc,
              xrows, srows, zbuf, acc, sem1, sem2, sem3, sem4):
        c = lax.axis_index("c")
        s = lax.axis_index("s")
        zero16 = jnp.zeros((16,), jnp.float32)
        iota16 = lax.iota(jnp.int32, 16)
        sent_row = jnp.zeros((16,), jnp.int32) + (RPB + s)
        sent_idx = iota16 + s * 16

        def zb_fill(i, _):
            for q in range(2 * F // 16):
                zbuf[i, pl.ds(q * 16, 16)] = zero16
            return 0
        lax.fori_loop(0, ZR, zb_fill, 0)

        def flush(cnt):
            def padf(j, _):
                o = jnp.minimum(cnt + j * 16, G)
                ckj[pl.ds(o, 16)] = sent_idx
                ct[pl.ds(o, 16)] = sent_idx
                cloc[pl.ds(o, 16)] = sent_row
                return 0
            lax.fori_loop(0, G // 16 + 1, padf, 0)

            d1 = pltpu.async_copy(xkj_hbm.at[ckj], xrows, sem1)
            d2 = pltpu.async_copy(sbf_hbm.at[ct], srows, sem2)
            d1.wait()
            d2.wait()

            def mult(r, _):
                for q in range(F // 16):
                    xrows[r, pl.ds(q * 16, 16)] = (
                        xrows[r, pl.ds(q * 16, 16)]
                        * srows[r, pl.ds(q * 16, 16)])
                return 0
            lax.fori_loop(0, GB, mult, 0)
            pltpu.sync_copy(xrows, acc.at[cloc], add=True)

        def one_pass(p, _):
            b = c * PASSES + p
            lo = b * RPB
            zd = [pltpu.async_copy(zbuf, acc.at[pl.ds(s * ZPT + z * ZR, ZR)],
                                   sem3) for z in range(NZ)]
            for d in zd:
                d.wait()
            plsc.subcore_barrier()

            bufs = ((jib0, kjb0, sem3), (jib1, kjb1, sem4))

            def chunk_fetch(ch, jib, kjb, semj):
                off = s * TSPAN + ch * CH
                dj = pltpu.async_copy(idxji_hbm.at[pl.ds(off, CH)], jib, semj)
                dk = pltpu.async_copy(idxkj_hbm.at[pl.ds(off, CH)], kjb, semj)
                return dj, dk

            def chunk_scan(ch, jib, kjb, cnt):
                off = s * TSPAN + ch * CH

                def compact(v, cnt):
                    vji = jib[pl.ds(v * 16, 16)]
                    vkj = kjb[pl.ds(v * 16, 16)]
                    msk = (vji >= lo) & (vji < lo + RPB)
                    loc = vji - lo
                    tv = off + v * 16 + iota16
                    plsc.store_compressed(ckj.at[pl.ds(cnt, 16)], vkj, mask=msk)
                    plsc.store_compressed(ct.at[pl.ds(cnt, 16)], tv, mask=msk)
                    plsc.store_compressed(cloc.at[pl.ds(cnt, 16)], loc, mask=msk)
                    cnt = cnt + plsc.all_reduce_population_count(msk)[0]

                    def do_flush(cc):
                        flush(cc)
                        return jnp.int32(0)
                    return lax.cond(cnt >= G - 15, do_flush,
                                    lambda cc: cc, cnt)
                return lax.fori_loop(0, NV, compact, cnt)

            cnt = jnp.int32(0)
            pend = chunk_fetch(0, *bufs[0])
            for ch in range(NCH):
                jib, kjb, _ = bufs[ch % 2]
                for d in pend:
                    d.wait()
                if ch + 1 < NCH:
                    pend = chunk_fetch(ch + 1, *bufs[(ch + 1) % 2])
                cnt = chunk_scan(ch, jib, kjb, cnt)
            flush(cnt)

            plsc.subcore_barrier()
            @pl.when(s < info_ns - 1)
            def _():
                pltpu.sync_copy(acc.at[pl.ds(s * CPT, CPT)],
                                out_hbm.at[pl.ds(b * RPB + s * CPT, CPT)])

            @pl.when(s == info_ns - 1)
            def _():
                pltpu.sync_copy(
                    acc.at[pl.ds((info_ns - 1) * CPT, CPL)],
                    out_hbm.at[pl.ds(b * RPB + (info_ns - 1) * CPT, CPL)])
            plsc.subcore_barrier()
            return 0
        lax.fori_loop(0, PASSES, one_pass, 0)

    return sc_fn


_sc_scatter = None


def _get_sc_scatter():
    global _sc_scatter
    if _sc_scatter is None:
        _sc_scatter = _make_sc_scatter(_E, _T, _F, NB=32, CH=2000, G=112)
    return _sc_scatter


def kernel(x, rbf, sbf, idx_kj, idx_ji, W_rbf1, W_rbf2, W_sbf1, W_sbf2,
           W_kj, b_kj, W_ji, b_ji, W_down, W_up,
           W_bs1, b_bs1, W_bs2, b_bs2, W_lin, b_lin,
           W_as1, b_as1, W_as2, b_as2, W_as3, b_as3, W_as4, b_as4):
    r2 = lambda v: v.reshape(1, -1)
    xji, xkj = _stage1(x, rbf, W_rbf1, W_rbf2, W_kj, r2(b_kj), W_ji,
                       r2(b_ji), W_down)
    sbf_e = _stage2(sbf, W_sbf1, W_sbf2)
    seg = _get_sc_scatter()(xkj, sbf_e, idx_kj.astype(jnp.int32),
                            idx_ji.astype(jnp.int32))
    return _stage4(seg, xji, x, W_up, W_bs1, r2(b_bs1), W_bs2, r2(b_bs2),
                   W_lin, r2(b_lin), W_as1, r2(b_as1), W_as2, r2(b_as2),
                   W_as3, r2(b_as3), W_as4, r2(b_as4))


# speedup vs baseline: 1.2474x; 1.0069x over previous
import functools

import jax
import jax.numpy as jnp
from jax import lax
from jax.experimental import pallas as pl
from jax.experimental.pallas import tpu as pltpu
from jax.experimental.pallas import tpu_sc as plsc

_E = 320000
_T = 640000
_H = 128
_F = 64


def _silu(v):
    return v * jax.nn.sigmoid(v)


def _bmm(a, w):
    return jax.lax.dot(a.astype(jnp.bfloat16), w.astype(jnp.bfloat16),
                       preferred_element_type=jnp.float32)


_BLK1 = 2560


def _s1_body(x_ref, rbf_ref, Wr1, Wr2, Wkj, bkj, Wji, bji, Wdn, xji_ref, xkj_ref):
    xb = x_ref[...]
    xji_ref[...] = _silu(_bmm(xb, Wji[...]) + bji[...])
    rbf_e = (rbf_ref[...] @ Wr1[...]) @ Wr2[...]
    t = _silu(_bmm(xb, Wkj[...]) + bkj[...]) * rbf_e
    v = _silu(_bmm(t, Wdn[...]))
    xkj_ref[...] = jnp.concatenate([v, jnp.zeros_like(v)], axis=1)


def _stage1(x, rbf, Wr1, Wr2, Wkj, bkj, Wji, bji, Wdn):
    nb = _E // _BLK1
    full = lambda a: pl.BlockSpec(a.shape, lambda i: (0,) * a.ndim)
    return pl.pallas_call(
        _s1_body,
        grid=(nb,),
        in_specs=[
            pl.BlockSpec((_BLK1, _H), lambda i: (i, 0)),
            pl.BlockSpec((_BLK1, 6), lambda i: (i, 0)),
            full(Wr1), full(Wr2), full(Wkj), full(bkj), full(Wji), full(bji),
            full(Wdn),
        ],
        out_specs=[
            pl.BlockSpec((_BLK1, _H), lambda i: (i, 0)),
            pl.BlockSpec((_BLK1, 2 * _F), lambda i: (i, 0)),
        ],
        out_shape=[
            jax.ShapeDtypeStruct((_E, _H), jnp.float32),
            jax.ShapeDtypeStruct((_E, 2 * _F), jnp.float32),
        ],
    )(x, rbf, Wr1, Wr2, Wkj, bkj, Wji, bji, Wdn)


_BLK2 = 2560


def _s2_body(sbf_ref, Ws1, Ws2, out_ref):
    r = (sbf_ref[...] @ Ws1[...]) @ Ws2[...]
    out_ref[...] = jnp.concatenate([r, jnp.zeros_like(r)], axis=1)


def _stage2(sbf, Ws1, Ws2):
    nb = _T // _BLK2
    full = lambda a: pl.BlockSpec(a.shape, lambda i: (0,) * a.ndim)
    return pl.pallas_call(
        _s2_body,
        grid=(nb,),
        in_specs=[
            pl.BlockSpec((_BLK2, 42), lambda i: (i, 0)),
            full(Ws1), full(Ws2),
        ],
        out_specs=pl.BlockSpec((_BLK2, 2 * _F), lambda i: (i, 0)),
        out_shape=jax.ShapeDtypeStruct((_T, 2 * _F), jnp.float32),
    )(sbf, Ws1, Ws2)


_BLK4 = 2560


def _s4_body(seg_ref, xji_ref, x_ref, Wup, Wb1, bb1, Wb2, bb2, Wl, bl,
             Wa1, ba1, Wa2, ba2, Wa3, ba3, Wa4, ba4, out_ref):
    h = xji_ref[...] + _silu(_bmm(seg_ref[...][:, :_F], Wup[...]))
    h = h + _silu(_bmm(_silu(_bmm(h, Wb1[...]) + bb1[...]), Wb2[...]) + bb2[...])
    h = _silu(_bmm(h, Wl[...]) + bl[...]) + x_ref[...]
    h = h + _silu(_bmm(_silu(_bmm(h, Wa1[...]) + ba1[...]), Wa2[...]) + ba2[...])
    h = h + _silu(_bmm(_silu(_bmm(h, Wa3[...]) + ba3[...]), Wa4[...]) + ba4[...])
    out_ref[...] = h


def _stage4(seg, xji, x, Wup, Wb1, bb1, Wb2, bb2, Wl, bl,
            Wa1, ba1, Wa2, ba2, Wa3, ba3, Wa4, ba4):
    nb = _E // _BLK4
    full = lambda a: pl.BlockSpec(a.shape, lambda i: (0,) * a.ndim)
    ws = [Wup, Wb1, bb1, Wb2, bb2, Wl, bl, Wa1, ba1, Wa2, ba2, Wa3, ba3, Wa4, ba4]
    return pl.pallas_call(
        _s4_body,
        grid=(nb,),
        in_specs=[
            pl.BlockSpec((_BLK4, 2 * _F), lambda i: (i, 0)),
            pl.BlockSpec((_BLK4, _H), lambda i: (i, 0)),
            pl.BlockSpec((_BLK4, _H), lambda i: (i, 0)),
        ] + [full(w) for w in ws],
        out_specs=pl.BlockSpec((_BLK4, _H), lambda i: (i, 0)),
        out_shape=jax.ShapeDtypeStruct((_E, _H), jnp.float32),
    )(seg, xji, x, *ws)


def _make_sc_scatter(E, T, F, NB, CH, G, interpret=False):
    info_nc, info_ns = 2, 16
    RPB = E // NB
    assert RPB * NB == E and RPB % 16 == 0
    PASSES = NB // info_nc
    ACC = RPB + 16
    while max(d for d in range(1, 97) if (ACC // info_ns) % d == 0) < 32:
        ACC += 16
    TSPAN = T // info_ns
    assert TSPAN % CH == 0
    NCH = TSPAN // CH
    assert CH % 16 == 0 and G % 16 == 0
    GB = G + 16
    NV = CH // 16
    ZPT = ACC // info_ns
    ZR = 32
    NZ = -(-ZPT // ZR)
    CPT = (RPB // info_ns) & ~7
    CPL = RPB - CPT * (info_ns - 1)
    assert CPL % 8 == 0 and CPL >= 0

    mesh = plsc.VectorSubcoreMesh(core_axis_name="c", subcore_axis_name="s",
                                  num_cores=info_nc, num_subcores=info_ns)

    @functools.partial(
        pl.kernel,
        out_type=jax.ShapeDtypeStruct((E, 2 * F), jnp.float32),
        mesh=mesh,
        interpret=interpret,
        compiler_params=pltpu.CompilerParams(needs_layout_passes=False),
        scratch_types=[
            pltpu.VMEM((CH,), jnp.int32),
            pltpu.VMEM((CH,), jnp.int32),
            pltpu.VMEM((CH,), jnp.int32),
            pltpu.VMEM((CH,), jnp.int32),
            pltpu.VMEM((GB,), jnp.int32),
            pltpu.VMEM((GB,), jnp.int32),
            pltpu.VMEM((GB,), jnp.int32),
            pltpu.VMEM((GB, 2 * F), jnp.float32),
            pltpu.VMEM((GB, 2 * F), jnp.float32),
            pltpu.VMEM((ZR, 2 * F), jnp.float32),
            pltpu.VMEM_SHARED((ACC, 2 * F), jnp.float32),
            pltpu.SemaphoreType.DMA,
            pltpu.SemaphoreType.DMA,
            pltpu.SemaphoreType.DMA,
            pltpu.SemaphoreType.DMA,
        ],
    )
    def sc_fn(xkj_hbm, sbf_hbm, idxkj_hbm, idxji_hbm, out_hbm,
              jib0, kjb0, jib1, kjb1, ckj, ct, cloc,
              xrows, srows, zbuf, acc, sem1, sem2, sem3, sem4):
        c = lax.axis_index("c")
        s = lax.axis_index("s")
        zero16 = jnp.zeros((16,), jnp.float32)
        iota16 = lax.iota(jnp.int32, 16)
        sent_row = jnp.zeros((16,), jnp.int32) + (RPB + s)
        sent_idx = iota16 + s * 16

        def zb_fill(i, _):
            for q in range(2 * F // 16):
                zbuf[i, pl.ds(q * 16, 16)] = zero16
            return 0
        lax.fori_loop(0, ZR, zb_fill, 0)

        def flush(cnt):
            def padf(j, _):
                o = jnp.minimum(cnt + j * 16, G)
                ckj[pl.ds(o, 16)] = sent_idx
                ct[pl.ds(o, 16)] = sent_idx
                cloc[pl.ds(o, 16)] = sent_row
                return 0
            lax.fori_loop(0, G // 16 + 1, padf, 0)

            d1 = pltpu.async_copy(xkj_hbm.at[ckj], xrows, sem1)
            d2 = pltpu.async_copy(sbf_hbm.at[ct], srows, sem2)
            d1.wait()
            d2.wait()

            def mult(r, _):
                for q in range(F // 16):
                    xrows[r, pl.ds(q * 16, 16)] = (
                        xrows[r, pl.ds(q * 16, 16)]
                        * srows[r, pl.ds(q * 16, 16)])
                return 0
            lax.fori_loop(0, GB, mult, 0)
            pltpu.sync_copy(xrows, acc.at[cloc], add=True)

        def one_pass(p, _):
            b = c * PASSES + p
            lo = b * RPB
            zd = [pltpu.async_copy(
                zbuf, acc.at[pl.ds(s * ZPT + min(z * ZR, ZPT - ZR), ZR)],
                sem3) for z in range(NZ)]
            for d in zd:
                d.wait()
            plsc.subcore_barrier()

            bufs = ((jib0, kjb0, sem3), (jib1, kjb1, sem4))

            def chunk_fetch(ch, jib, kjb, semj):
                off = s * TSPAN + ch * CH
                dj = pltpu.async_copy(idxji_hbm.at[pl.ds(off, CH)], jib, semj)
                dk = pltpu.async_copy(idxkj_hbm.at[pl.ds(off, CH)], kjb, semj)
                return dj, dk

            def chunk_scan(ch, jib, kjb, cnt):
                off = s * TSPAN + ch * CH

                def compact(v, cnt):
                    vji = jib[pl.ds(v * 16, 16)]
                    vkj = kjb[pl.ds(v * 16, 16)]
                    msk = (vji >= lo) & (vji < lo + RPB)
                    loc = vji - lo
                    tv = off + v * 16 + iota16
                    plsc.store_compressed(ckj.at[pl.ds(cnt, 16)], vkj, mask=msk)
                    plsc.store_compressed(ct.at[pl.ds(cnt, 16)], tv, mask=msk)
                    plsc.store_compressed(cloc.at[pl.ds(cnt, 16)], loc, mask=msk)
                    cnt = cnt + plsc.all_reduce_population_count(msk)[0]

                    def do_flush(cc):
                        flush(cc)
                        return jnp.int32(0)
                    return lax.cond(cnt >= G - 15, do_flush,
                                    lambda cc: cc, cnt)
                return lax.fori_loop(0, NV, compact, cnt)

            cnt = jnp.int32(0)
            pend = chunk_fetch(0, *bufs[0])
            for ch in range(NCH):
                jib, kjb, _ = bufs[ch % 2]
                for d in pend:
                    d.wait()
                if ch + 1 < NCH:
                    pend = chunk_fetch(ch + 1, *bufs[(ch + 1) % 2])
                cnt = chunk_scan(ch, jib, kjb, cnt)
            flush(cnt)

            plsc.subcore_barrier()
            @pl.when(s < info_ns - 1)
            def _():
                pltpu.sync_copy(acc.at[pl.ds(s * CPT, CPT)],
                                out_hbm.at[pl.ds(b * RPB + s * CPT, CPT)])

            @pl.when(s == info_ns - 1)
            def _():
                pltpu.sync_copy(
                    acc.at[pl.ds((info_ns - 1) * CPT, CPL)],
                    out_hbm.at[pl.ds(b * RPB + (info_ns - 1) * CPT, CPL)])
            plsc.subcore_barrier()
            return 0
        lax.fori_loop(0, PASSES, one_pass, 0)

    return sc_fn


_sc_scatter = None


def _get_sc_scatter():
    global _sc_scatter
    if _sc_scatter is None:
        _sc_scatter = _make_sc_scatter(_E, _T, _F, NB=32, CH=2000, G=128)
    return _sc_scatter


def kernel(x, rbf, sbf, idx_kj, idx_ji, W_rbf1, W_rbf2, W_sbf1, W_sbf2,
           W_kj, b_kj, W_ji, b_ji, W_down, W_up,
           W_bs1, b_bs1, W_bs2, b_bs2, W_lin, b_lin,
           W_as1, b_as1, W_as2, b_as2, W_as3, b_as3, W_as4, b_as4):
    r2 = lambda v: v.reshape(1, -1)
    xji, xkj = _stage1(x, rbf, W_rbf1, W_rbf2, W_kj, r2(b_kj), W_ji,
                       r2(b_ji), W_down)
    sbf_e = _stage2(sbf, W_sbf1, W_sbf2)
    seg = _get_sc_scatter()(xkj, sbf_e, idx_kj.astype(jnp.int32),
                            idx_ji.astype(jnp.int32))
    return _stage4(seg, xji, x, W_up, W_bs1, r2(b_bs1), W_bs2, r2(b_bs2),
                   W_lin, r2(b_lin), W_as1, r2(b_as1), W_as2, r2(b_as2),
                   W_as3, r2(b_as3), W_as4, r2(b_as4))
